# Initial kernel scaffold; baseline (speedup 1.0000x reference)
import jax
import jax.numpy as jnp
from jax.experimental import pallas as pl


def kernel(x, edge_index, batch, c1_W, c1_as, c1_ad, c1_b, c2_W, c2_as, c2_ad, c2_b, c3_W, c3_as, c3_ad, c3_b, c4_W, c4_as, c4_ad, c4_b, c5_W, c5_as, c5_ad, c5_b, md1_W, md1_b, md2_W, md2_b, mh1_W, mh1_b, mh2_W, mh2_b, ln_w, ln_b):
    raise NotImplementedError("write your pallas kernel here")


# speedup vs baseline: 1.0257x; 1.0257x over previous
import functools

import jax
import jax.numpy as jnp
import numpy as np
from jax import lax
from jax.experimental import pallas as pl
from jax.experimental.pallas import tpu as pltpu

H_ = 5
C_ = 64
HC = 320
G_ = 64
NB_B = 1000


def _head_maps():
    Mh = np.zeros((2, 160, 16), np.float32)
    Me = np.zeros((2, 16, 160), np.float32)
    for j in range(2):
        for c in range(160):
            h = (j * 160 + c) // 64
            Mh[j, c, h] = 1.0
            Me[j, h, c] = 1.0
    return jnp.asarray(Mh), jnp.asarray(Me)


def _tca_body(x_ref, W_ref, asv_ref, adv_ref, M_ref, h2_ref, als_ref, ald_ref):
    j = pl.program_id(1)
    hb = jnp.dot(x_ref[...], W_ref[0], preferred_element_type=jnp.float32)
    h2_ref[0] = hb
    ps = jnp.dot(hb * asv_ref[0], M_ref[0], preferred_element_type=jnp.float32)
    pd = jnp.dot(hb * adv_ref[0], M_ref[0], preferred_element_type=jnp.float32)

    @pl.when(j == 0)
    def _():
        als_ref[...] = ps
        ald_ref[...] = pd

    @pl.when(j != 0)
    def _():
        als_ref[...] += ps
        ald_ref[...] += pd


def _tca(x, W, asv, adv, Mh):
    N, Din = x.shape
    B = NB_B
    W2 = W.reshape(Din, 2, 160).transpose(1, 0, 2)
    return pl.pallas_call(
        _tca_body,
        grid=(N // B, 2),
        in_specs=[
            pl.BlockSpec((B, Din), lambda i, j: (i, 0)),
            pl.BlockSpec((1, Din, 160), lambda i, j: (j, 0, 0)),
            pl.BlockSpec((1, 1, 160), lambda i, j: (j, 0, 0)),
            pl.BlockSpec((1, 1, 160), lambda i, j: (j, 0, 0)),
            pl.BlockSpec((1, 160, 16), lambda i, j: (j, 0, 0)),
        ],
        out_specs=[
            pl.BlockSpec((1, B, 160), lambda i, j: (j, i, 0)),
            pl.BlockSpec((B, 16), lambda i, j: (i, 0)),
            pl.BlockSpec((B, 16), lambda i, j: (i, 0)),
        ],
        out_shape=[
            jax.ShapeDtypeStruct((2, N, 160), jnp.float32),
            jax.ShapeDtypeStruct((N, 16), jnp.float32),
            jax.ShapeDtypeStruct((N, 16), jnp.float32),
        ],
    )(x, W2, asv, adv, Mh)


def _tcb_body_mlp(accA_r, accB_r, denA_r, denB_r, b_r, x_r, mdW_r, mdb_r,
                  Me_r, out_r):
    den16 = denA_r[...] + denB_r[...]
    dA = jnp.dot(den16, Me_r[0], preferred_element_type=jnp.float32)
    dB = jnp.dot(den16, Me_r[1], preferred_element_type=jnp.float32)
    msg = jnp.concatenate([accA_r[...] / dA, accB_r[...] / dB], axis=-1)
    x0 = jnp.maximum(msg + b_r[...], 0.0)
    out_r[...] = (jnp.dot(x_r[...], mdW_r[...],
                          preferred_element_type=jnp.float32)
                  + mdb_r[...] + x0)


def _tcb_body_plain(accA_r, accB_r, denA_r, denB_r, b_r, x_r, Me_r, out_r):
    den16 = denA_r[...] + denB_r[...]
    dA = jnp.dot(den16, Me_r[0], preferred_element_type=jnp.float32)
    dB = jnp.dot(den16, Me_r[1], preferred_element_type=jnp.float32)
    msg = jnp.concatenate([accA_r[...] / dA, accB_r[...] / dB], axis=-1)
    x0 = jnp.maximum(msg + b_r[...], 0.0)
    out_r[...] = x_r[...] + x0


def _tcb(accA, accB, denA, denB, b, x, Me, mdW=None, mdb=None):
    N = accA.shape[0]
    B = NB_B
    Din = x.shape[1]
    common_in = [
        pl.BlockSpec((B, 160), lambda i: (i, 0)),
        pl.BlockSpec((B, 160), lambda i: (i, 0)),
        pl.BlockSpec((B, 16), lambda i: (i, 0)),
        pl.BlockSpec((B, 16), lambda i: (i, 0)),
        pl.BlockSpec((1, HC), lambda i: (0, 0)),
        pl.BlockSpec((B, Din), lambda i: (i, 0)),
    ]
    if mdW is not None:
        in_specs = common_in + [
            pl.BlockSpec((Din, HC), lambda i: (0, 0)),
            pl.BlockSpec((1, HC), lambda i: (0, 0)),
            pl.BlockSpec((2, 16, 160), lambda i: (0, 0, 0)),
        ]
        args = (accA, accB, denA, denB, b.reshape(1, HC), x, mdW,
                mdb.reshape(1, HC), Me)
        body = _tcb_body_mlp
    else:
        in_specs = common_in + [pl.BlockSpec((2, 16, 160), lambda i: (0, 0, 0))]
        args = (accA, accB, denA, denB, b.reshape(1, HC), x, Me)
        body = _tcb_body_plain
    return pl.pallas_call(
        body,
        grid=(N // B,),
        in_specs=in_specs,
        out_specs=pl.BlockSpec((B, HC), lambda i: (i, 0)),
        out_shape=jax.ShapeDtypeStruct((N, HC), jnp.float32),
    )(*args)


def _pool_body(bf_r, x_r, s_r, cnt_r):
    i = pl.program_id(0)
    gids = lax.broadcasted_iota(jnp.int32, (1, G_), 1).astype(jnp.float32)
    mask = (bf_r[...] == gids).astype(jnp.float32)
    sp = lax.dot_general(mask, x_r[...], (((0,), (0,)), ((), ())),
                         preferred_element_type=jnp.float32)
    cp = lax.dot_general(mask, jnp.ones((mask.shape[0], 128), jnp.float32),
                         (((0,), (0,)), ((), ())),
                         preferred_element_type=jnp.float32)

    @pl.when(i == 0)
    def _():
        s_r[...] = sp
        cnt_r[...] = cp

    @pl.when(i != 0)
    def _():
        s_r[...] += sp
        cnt_r[...] += cp


def _pool(bf, x):
    N = x.shape[0]
    B = NB_B
    return pl.pallas_call(
        _pool_body,
        grid=(N // B,),
        in_specs=[
            pl.BlockSpec((B, 1), lambda i: (i, 0)),
            pl.BlockSpec((B, HC), lambda i: (i, 0)),
        ],
        out_specs=[
            pl.BlockSpec((G_, HC), lambda i: (0, 0)),
            pl.BlockSpec((G_, 128), lambda i: (0, 0)),
        ],
        out_shape=[
            jax.ShapeDtypeStruct((G_, HC), jnp.float32),
            jax.ShapeDtypeStruct((G_, 128), jnp.float32),
        ],
    )(bf, x)


def _head_body(s_r, cnt_r, w1_r, b1_r, w2_r, b2_r, lnw_r, lnb_r, out_r):
    cnt = jnp.maximum(cnt_r[:, 0:1], 1.0)
    xg = s_r[...] / cnt
    h1 = jnp.maximum(jnp.dot(xg, w1_r[...],
                             preferred_element_type=jnp.float32) + b1_r[...],
                     0.0)
    o = jnp.dot(h1, w2_r[...], preferred_element_type=jnp.float32) + b2_r[...]
    mu = jnp.mean(o, axis=-1, keepdims=True)
    var = jnp.mean((o - mu) ** 2, axis=-1, keepdims=True)
    out_r[...] = (o - mu) / jnp.sqrt(var + 1e-5) * lnw_r[...] + lnb_r[...]


def _headmlp(s, cnt, w1, b1, w2, b2, lnw, lnb):
    nhid = w1.shape[1]
    nout = w2.shape[1]
    return pl.pallas_call(
        _head_body,
        out_shape=jax.ShapeDtypeStruct((G_, nout), jnp.float32),
    )(s, cnt, w1, b1.reshape(1, nhid), w2, b2.reshape(1, nout),
      lnw.reshape(1, nout), lnb.reshape(1, nout))


def _edge_jax(h2, alsp, aldp, src, dst):
    N = alsp.shape[0]
    als = alsp[:, :H_]
    ald = aldp[:, :H_]
    e = als[src] + ald[dst]
    w = jnp.exp(jnp.maximum(e, 0.2 * e))
    denh = jax.ops.segment_sum(w, dst, num_segments=N)
    h = jnp.concatenate([h2[0], h2[1]], axis=1)
    msg = h[src].reshape(-1, H_, C_) * w[:, :, None]
    acc = jax.ops.segment_sum(msg, dst, num_segments=N).reshape(N, HC)
    accA, accB = acc[:, :160], acc[:, 160:]
    z = jnp.zeros((N, 1), jnp.float32)
    denA = jnp.concatenate([denh[:, :3]] + [z] * 13, axis=1)
    denB = jnp.concatenate([z] * 3 + [denh[:, 3:5]] + [z] * 11, axis=1)
    return accA, accB, denA, denB


def kernel(x, edge_index, batch, c1_W, c1_as, c1_ad, c1_b, c2_W, c2_as, c2_ad,
           c2_b, c3_W, c3_as, c3_ad, c3_b, c4_W, c4_as, c4_ad, c4_b, c5_W,
           c5_as, c5_ad, c5_b, md1_W, md1_b, md2_W, md2_b, mh1_W, mh1_b,
           mh2_W, mh2_b, ln_w, ln_b):
    N = x.shape[0]
    loop = jnp.arange(N, dtype=edge_index.dtype)
    src = jnp.concatenate([edge_index[0], loop])
    dst = jnp.concatenate([edge_index[1], loop])
    Mh, Me = _head_maps()

    layers = [
        (c1_W, c1_as, c1_ad, c1_b, md1_W, md1_b),
        (c2_W, c2_as, c2_ad, c2_b, md2_W, md2_b),
        (c3_W, c3_as, c3_ad, c3_b, None, None),
        (c4_W, c4_as, c4_ad, c4_b, None, None),
        (c5_W, c5_as, c5_ad, c5_b, None, None),
    ]

    xcur = x
    for (W, a_s, a_d, b, mdW, mdb) in layers:
        asv = a_s.reshape(2, 1, 160)
        adv = a_d.reshape(2, 1, 160)
        h2, alsp, aldp = _tca(xcur, W, asv, adv, Mh)
        accA, accB, denA, denB = _edge_jax(h2, alsp, aldp, src, dst)
        xcur = _tcb(accA, accB, denA, denB, b, xcur, Me, mdW, mdb)

    bf = batch.astype(jnp.float32).reshape(N, 1)
    s, cnt = _pool(bf, xcur)
    return _headmlp(s, cnt, mh1_W, mh1_b, mh2_W, mh2_b, ln_w, ln_b)


# speedup vs baseline: 13.5365x; 13.1970x over previous
import functools

import jax
import jax.numpy as jnp
import numpy as np
from jax import lax
from jax.experimental import pallas as pl
from jax.experimental.pallas import tpu as pltpu

H_ = 5
C_ = 64
HC = 320
G_ = 64
NB_B = 1000


def _head_maps():
    Mh = np.zeros((2, 160, 16), np.float32)
    Me = np.zeros((2, 16, 160), np.float32)
    for j in range(2):
        for c in range(160):
            h = (j * 160 + c) // 64
            Mh[j, c, h] = 1.0
            Me[j, h, c] = 1.0
    return jnp.asarray(Mh), jnp.asarray(Me)


def _tca_body(x_ref, W_ref, asv_ref, adv_ref, M_ref, h2_ref, als_ref, ald_ref):
    j = pl.program_id(1)
    hb = jnp.dot(x_ref[...], W_ref[0], preferred_element_type=jnp.float32)
    h2_ref[0] = hb
    ps = jnp.dot(hb * asv_ref[0], M_ref[0], preferred_element_type=jnp.float32)
    pd = jnp.dot(hb * adv_ref[0], M_ref[0], preferred_element_type=jnp.float32)

    @pl.when(j == 0)
    def _():
        als_ref[...] = ps
        ald_ref[...] = pd

    @pl.when(j != 0)
    def _():
        als_ref[...] += ps
        ald_ref[...] += pd


def _tca(x, W, asv, adv, Mh):
    N, Din = x.shape
    B = NB_B
    W2 = W.reshape(Din, 2, 160).transpose(1, 0, 2)
    return pl.pallas_call(
        _tca_body,
        grid=(N // B, 2),
        in_specs=[
            pl.BlockSpec((B, Din), lambda i, j: (i, 0)),
            pl.BlockSpec((1, Din, 160), lambda i, j: (j, 0, 0)),
            pl.BlockSpec((1, 1, 160), lambda i, j: (j, 0, 0)),
            pl.BlockSpec((1, 1, 160), lambda i, j: (j, 0, 0)),
            pl.BlockSpec((1, 160, 16), lambda i, j: (j, 0, 0)),
        ],
        out_specs=[
            pl.BlockSpec((1, B, 160), lambda i, j: (j, i, 0)),
            pl.BlockSpec((B, 16), lambda i, j: (i, 0)),
            pl.BlockSpec((B, 16), lambda i, j: (i, 0)),
        ],
        out_shape=[
            jax.ShapeDtypeStruct((2, N, 160), jnp.float32),
            jax.ShapeDtypeStruct((N, 16), jnp.float32),
            jax.ShapeDtypeStruct((N, 16), jnp.float32),
        ],
    )(x, W2, asv, adv, Mh)


def _tcb_body_mlp(accA_r, accB_r, denA_r, denB_r, b_r, x_r, mdW_r, mdb_r,
                  Me_r, out_r):
    den16 = denA_r[...] + denB_r[...]
    dA = jnp.dot(den16, Me_r[0], preferred_element_type=jnp.float32)
    dB = jnp.dot(den16, Me_r[1], preferred_element_type=jnp.float32)
    msg = jnp.concatenate([accA_r[...] / dA, accB_r[...] / dB], axis=-1)
    x0 = jnp.maximum(msg + b_r[...], 0.0)
    out_r[...] = (jnp.dot(x_r[...], mdW_r[...],
                          preferred_element_type=jnp.float32)
                  + mdb_r[...] + x0)


def _tcb_body_plain(accA_r, accB_r, denA_r, denB_r, b_r, x_r, Me_r, out_r):
    den16 = denA_r[...] + denB_r[...]
    dA = jnp.dot(den16, Me_r[0], preferred_element_type=jnp.float32)
    dB = jnp.dot(den16, Me_r[1], preferred_element_type=jnp.float32)
    msg = jnp.concatenate([accA_r[...] / dA, accB_r[...] / dB], axis=-1)
    x0 = jnp.maximum(msg + b_r[...], 0.0)
    out_r[...] = x_r[...] + x0


def _tcb(accA, accB, denA, denB, b, x, Me, mdW=None, mdb=None):
    N = accA.shape[0]
    B = NB_B
    Din = x.shape[1]
    common_in = [
        pl.BlockSpec((B, 160), lambda i: (i, 0)),
        pl.BlockSpec((B, 160), lambda i: (i, 0)),
        pl.BlockSpec((B, 16), lambda i: (i, 0)),
        pl.BlockSpec((B, 16), lambda i: (i, 0)),
        pl.BlockSpec((1, HC), lambda i: (0, 0)),
        pl.BlockSpec((B, Din), lambda i: (i, 0)),
    ]
    if mdW is not None:
        in_specs = common_in + [
            pl.BlockSpec((Din, HC), lambda i: (0, 0)),
            pl.BlockSpec((1, HC), lambda i: (0, 0)),
            pl.BlockSpec((2, 16, 160), lambda i: (0, 0, 0)),
        ]
        args = (accA, accB, denA, denB, b.reshape(1, HC), x, mdW,
                mdb.reshape(1, HC), Me)
        body = _tcb_body_mlp
    else:
        in_specs = common_in + [pl.BlockSpec((2, 16, 160), lambda i: (0, 0, 0))]
        args = (accA, accB, denA, denB, b.reshape(1, HC), x, Me)
        body = _tcb_body_plain
    return pl.pallas_call(
        body,
        grid=(N // B,),
        in_specs=in_specs,
        out_specs=pl.BlockSpec((B, HC), lambda i: (i, 0)),
        out_shape=jax.ShapeDtypeStruct((N, HC), jnp.float32),
    )(*args)


def _pool_body(bf_r, x_r, s_r, cnt_r):
    i = pl.program_id(0)
    gids = lax.broadcasted_iota(jnp.int32, (1, G_), 1).astype(jnp.float32)
    mask = (bf_r[...] == gids).astype(jnp.float32)
    sp = lax.dot_general(mask, x_r[...], (((0,), (0,)), ((), ())),
                         preferred_element_type=jnp.float32)
    cp = lax.dot_general(mask, jnp.ones((mask.shape[0], 128), jnp.float32),
                         (((0,), (0,)), ((), ())),
                         preferred_element_type=jnp.float32)

    @pl.when(i == 0)
    def _():
        s_r[...] = sp
        cnt_r[...] = cp

    @pl.when(i != 0)
    def _():
        s_r[...] += sp
        cnt_r[...] += cp


def _pool(bf, x):
    N = x.shape[0]
    B = NB_B
    return pl.pallas_call(
        _pool_body,
        grid=(N // B,),
        in_specs=[
            pl.BlockSpec((B, 1), lambda i: (i, 0)),
            pl.BlockSpec((B, HC), lambda i: (i, 0)),
        ],
        out_specs=[
            pl.BlockSpec((G_, HC), lambda i: (0, 0)),
            pl.BlockSpec((G_, 128), lambda i: (0, 0)),
        ],
        out_shape=[
            jax.ShapeDtypeStruct((G_, HC), jnp.float32),
            jax.ShapeDtypeStruct((G_, 128), jnp.float32),
        ],
    )(bf, x)


def _head_body(s_r, cnt_r, w1_r, b1_r, w2_r, b2_r, lnw_r, lnb_r, out_r):
    cnt = jnp.maximum(cnt_r[:, 0:1], 1.0)
    xg = s_r[...] / cnt
    h1 = jnp.maximum(jnp.dot(xg, w1_r[...],
                             preferred_element_type=jnp.float32) + b1_r[...],
                     0.0)
    o = jnp.dot(h1, w2_r[...], preferred_element_type=jnp.float32) + b2_r[...]
    mu = jnp.mean(o, axis=-1, keepdims=True)
    var = jnp.mean((o - mu) ** 2, axis=-1, keepdims=True)
    out_r[...] = (o - mu) / jnp.sqrt(var + 1e-5) * lnw_r[...] + lnb_r[...]


def _headmlp(s, cnt, w1, b1, w2, b2, lnw, lnb):
    nhid = w1.shape[1]
    nout = w2.shape[1]
    return pl.pallas_call(
        _head_body,
        out_shape=jax.ShapeDtypeStruct((G_, nout), jnp.float32),
    )(s, cnt, w1, b1.reshape(1, nhid), w2, b2.reshape(1, nout),
      lnw.reshape(1, nout), lnb.reshape(1, nout))


NP_PAD = 10240
E_K = 32


@functools.lru_cache(maxsize=None)
def _sc_edge_build(N, Epad):
    from jax.experimental.pallas import tpu_sc as plsc

    NSUB = 16
    EPS = Epad // NSUB
    NCHUNK = EPS // E_K
    RPS = NP_PAD // NSUB

    mesh = plsc.VectorSubcoreMesh(core_axis_name="c", subcore_axis_name="s")

    def body(h2_r, als_r, ald_r, src_r, dst_r, acc_out, den_out,
             acc_sh, den_sh, src_v, dst_v, gidx_v, rows_v, as_v, ad_v,
             w_v, dv_v, zb_v, z16_v, tmp_v, t16_v, sem):
        c = lax.axis_index("c")
        s = lax.axis_index("s")
        lane = lax.broadcasted_iota(jnp.int32, (16,), 0)
        zero = jnp.zeros((16,), jnp.float32)
        cb = jnp.zeros((16,), jnp.int32) + c
        is0 = cb == 0
        den_mask = jnp.where(is0, lane < 3, (lane >= 3) & (lane < 5))
        headv = [jnp.where(is0,
                           jnp.full((16,), v // 4, jnp.int32),
                           jnp.full((16,), (10 + v) // 4, jnp.int32))
                 for v in range(10)]
        cN16 = cb * N

        for r in range(16):
            for v in range(10):
                zb_v[r, pl.ds(v * 16, 16)] = zero
            z16_v[r] = zero

        def zbody(k, carry):
            r0 = s * RPS + k * 16
            pltpu.sync_copy(zb_v, acc_sh.at[pl.ds(r0, 16)])
            pltpu.sync_copy(z16_v, den_sh.at[pl.ds(r0, 16)])
            return carry

        lax.fori_loop(0, RPS // 16, zbody, 0)
        plsc.subcore_barrier()

        def ebody(i, carry):
            base = s * EPS + i * E_K
            pltpu.sync_copy(src_r.at[pl.ds(base, E_K)], src_v)
            pltpu.sync_copy(dst_r.at[pl.ds(base, E_K)], dst_v)
            for jj in range(E_K // 16):
                gidx_v[pl.ds(jj * 16, 16)] = src_v[pl.ds(jj * 16, 16)] + cN16
            pltpu.async_copy(h2_r.at[gidx_v], rows_v, sem).wait()
            pltpu.async_copy(als_r.at[src_v], as_v, sem).wait()
            pltpu.async_copy(ald_r.at[dst_v], ad_v, sem).wait()
            for j in range(E_K):
                e = as_v[j] + ad_v[j]
                w = jnp.exp(jnp.maximum(e, 0.2 * e))
                w_v[pl.ds(j * 16, 16)] = w
                dv_v[j] = jnp.where(den_mask, w, 0.0)
            for j in range(E_K):
                for v in range(10):
                    splat = plsc.load_gather(w_v, [headv[v] + (j * 16)])
                    r = rows_v[j, pl.ds(v * 16, 16)]
                    rows_v[j, pl.ds(v * 16, 16)] = r * splat
            pltpu.sync_copy(rows_v, acc_sh.at[dst_v], add=True)
            pltpu.sync_copy(dv_v, den_sh.at[dst_v], add=True)
            return carry

        lax.fori_loop(0, NCHUNK, ebody, 0)
        plsc.subcore_barrier()

        def wbody(k, carry):
            r0 = s * RPS + k * 16
            ro = c * NP_PAD + r0
            pltpu.sync_copy(acc_sh.at[pl.ds(r0, 16)], tmp_v)
            pltpu.sync_copy(tmp_v, acc_out.at[pl.ds(ro, 16)])
            pltpu.sync_copy(den_sh.at[pl.ds(r0, 16)], t16_v)
            pltpu.sync_copy(t16_v, den_out.at[pl.ds(ro, 16)])
            return carry

        lax.fori_loop(0, RPS // 16, wbody, 0)

    return pl.kernel(
        body,
        out_type=[
            jax.ShapeDtypeStruct((2 * NP_PAD, 160), jnp.float32),
            jax.ShapeDtypeStruct((2 * NP_PAD, 16), jnp.float32),
        ],
        mesh=mesh,
        compiler_params=pltpu.CompilerParams(needs_layout_passes=False,
                                             use_tc_tiling_on_sc=False),
        scratch_types=[
            pltpu.VMEM_SHARED((NP_PAD, 160), jnp.float32),
            pltpu.VMEM_SHARED((NP_PAD, 16), jnp.float32),
            pltpu.VMEM((E_K,), jnp.int32),
            pltpu.VMEM((E_K,), jnp.int32),
            pltpu.VMEM((E_K,), jnp.int32),
            pltpu.VMEM((E_K, 160), jnp.float32),
            pltpu.VMEM((E_K, 16), jnp.float32),
            pltpu.VMEM((E_K, 16), jnp.float32),
            pltpu.VMEM((E_K * 16,), jnp.float32),
            pltpu.VMEM((E_K, 16), jnp.float32),
            pltpu.VMEM((16, 160), jnp.float32),
            pltpu.VMEM((16, 16), jnp.float32),
            pltpu.VMEM((16, 160), jnp.float32),
            pltpu.VMEM((16, 16), jnp.float32),
            pltpu.SemaphoreType.DMA,
        ],
    )


def _edge_sc(h2, alsp, aldp, srcp, dstp):
    N = alsp.shape[0]
    Epad = srcp.shape[0]
    h2f = h2.reshape(2 * N, 160)
    acc2, den2 = _sc_edge_build(N, Epad)(h2f, alsp, aldp, srcp, dstp)
    acc2 = acc2.reshape(2, NP_PAD, 160)
    den2 = den2.reshape(2, NP_PAD, 16)
    return acc2[0, :N], acc2[1, :N], den2[0, :N], den2[1, :N]


def _edge_jax(h2, alsp, aldp, src, dst):
    N = alsp.shape[0]
    als = alsp[:, :H_]
    ald = aldp[:, :H_]
    e = als[src] + ald[dst]
    w = jnp.exp(jnp.maximum(e, 0.2 * e))
    denh = jax.ops.segment_sum(w, dst, num_segments=N)
    h = jnp.concatenate([h2[0], h2[1]], axis=1)
    msg = h[src].reshape(-1, H_, C_) * w[:, :, None]
    acc = jax.ops.segment_sum(msg, dst, num_segments=N).reshape(N, HC)
    accA, accB = acc[:, :160], acc[:, 160:]
    z = jnp.zeros((N, 1), jnp.float32)
    denA = jnp.concatenate([denh[:, :3]] + [z] * 13, axis=1)
    denB = jnp.concatenate([z] * 3 + [denh[:, 3:5]] + [z] * 11, axis=1)
    return accA, accB, denA, denB


def kernel(x, edge_index, batch, c1_W, c1_as, c1_ad, c1_b, c2_W, c2_as, c2_ad,
           c2_b, c3_W, c3_as, c3_ad, c3_b, c4_W, c4_as, c4_ad, c4_b, c5_W,
           c5_as, c5_ad, c5_b, md1_W, md1_b, md2_W, md2_b, mh1_W, mh1_b,
           mh2_W, mh2_b, ln_w, ln_b):
    N = x.shape[0]
    loop = jnp.arange(N, dtype=edge_index.dtype)
    E_tot = edge_index.shape[1] + N
    Epad = ((E_tot + 16 * E_K - 1) // (16 * E_K)) * (16 * E_K)
    padn = Epad - E_tot
    srcp = jnp.concatenate(
        [edge_index[0], loop, jnp.zeros((padn,), edge_index.dtype)])
    dstp = jnp.concatenate(
        [edge_index[1], loop,
         jnp.full((padn,), NP_PAD - 1, edge_index.dtype)])
    Mh, Me = _head_maps()

    layers = [
        (c1_W, c1_as, c1_ad, c1_b, md1_W, md1_b),
        (c2_W, c2_as, c2_ad, c2_b, md2_W, md2_b),
        (c3_W, c3_as, c3_ad, c3_b, None, None),
        (c4_W, c4_as, c4_ad, c4_b, None, None),
        (c5_W, c5_as, c5_ad, c5_b, None, None),
    ]

    xcur = x
    for (W, a_s, a_d, b, mdW, mdb) in layers:
        asv = a_s.reshape(2, 1, 160)
        adv = a_d.reshape(2, 1, 160)
        h2, alsp, aldp = _tca(xcur, W, asv, adv, Mh)
        accA, accB, denA, denB = _edge_sc(h2, alsp, aldp, srcp, dstp)
        xcur = _tcb(accA, accB, denA, denB, b, xcur, Me, mdW, mdb)

    bf = batch.astype(jnp.float32).reshape(N, 1)
    s, cnt = _pool(bf, xcur)
    return _headmlp(s, cnt, mh1_W, mh1_b, mh2_W, mh2_b, ln_w, ln_b)


# speedup vs baseline: 17.4625x; 1.2900x over previous
import functools

import jax
import jax.numpy as jnp
import numpy as np
from jax import lax
from jax.experimental import pallas as pl
from jax.experimental.pallas import tpu as pltpu

H_ = 5
C_ = 64
HC = 320
G_ = 64
NB_B = 1000
NP_PAD = 10016
E_K = 16


def _head_maps():
    Mh = np.zeros((320, 16), np.float32)
    Me = np.zeros((2, 16, 160), np.float32)
    for c in range(320):
        h = c // 64
        Mh[c, h] = 1.0
        Me[c // 160, h, c % 160] = 1.0
    return jnp.asarray(Mh), jnp.asarray(Me)


def _tca_body(x_ref, W_ref, asv_ref, adv_ref, M_ref, h2_ref, ald_ref):
    hb = jnp.dot(x_ref[...], W_ref[...], preferred_element_type=jnp.float32)
    als = jnp.dot(hb * asv_ref[...], M_ref[...],
                  preferred_element_type=jnp.float32)
    ald_ref[...] = jnp.dot(hb * adv_ref[...], M_ref[...],
                           preferred_element_type=jnp.float32)
    blk0 = jnp.concatenate([hb[:, :160], als], axis=-1)
    blk1 = jnp.concatenate([hb[:, 160:], als], axis=-1)
    h2_ref[...] = jnp.stack([blk0, blk1], axis=0)


def _tca(x, W, asv, adv, Mh):
    N, Din = x.shape
    B = NB_B
    return pl.pallas_call(
        _tca_body,
        grid=(N // B,),
        in_specs=[
            pl.BlockSpec((B, Din), lambda i: (i, 0)),
            pl.BlockSpec((Din, HC), lambda i: (0, 0)),
            pl.BlockSpec((1, HC), lambda i: (0, 0)),
            pl.BlockSpec((1, HC), lambda i: (0, 0)),
            pl.BlockSpec((HC, 16), lambda i: (0, 0)),
        ],
        out_specs=[
            pl.BlockSpec((2, B, 176), lambda i: (0, i, 0)),
            pl.BlockSpec((B, 16), lambda i: (i, 0)),
        ],
        out_shape=[
            jax.ShapeDtypeStruct((2, N, 176), jnp.float32),
            jax.ShapeDtypeStruct((N, 16), jnp.float32),
        ],
    )(x, W, asv, adv, Mh)


def _tcb_body_mlp(accA_r, accB_r, denA_r, denB_r, b_r, x_r, mdW_r, mdb_r,
                  Me_r, out_r):
    den16 = denA_r[...] + denB_r[...]
    dA = jnp.dot(den16, Me_r[0], preferred_element_type=jnp.float32)
    dB = jnp.dot(den16, Me_r[1], preferred_element_type=jnp.float32)
    msg = jnp.concatenate([accA_r[...] / dA, accB_r[...] / dB], axis=-1)
    x0 = jnp.maximum(msg + b_r[...], 0.0)
    out_r[...] = (jnp.dot(x_r[...], mdW_r[...],
                          preferred_element_type=jnp.float32)
                  + mdb_r[...] + x0)


def _tcb_body_plain(accA_r, accB_r, denA_r, denB_r, b_r, x_r, Me_r, out_r):
    den16 = denA_r[...] + denB_r[...]
    dA = jnp.dot(den16, Me_r[0], preferred_element_type=jnp.float32)
    dB = jnp.dot(den16, Me_r[1], preferred_element_type=jnp.float32)
    msg = jnp.concatenate([accA_r[...] / dA, accB_r[...] / dB], axis=-1)
    x0 = jnp.maximum(msg + b_r[...], 0.0)
    out_r[...] = x_r[...] + x0


def _tcb(accA, accB, denA, denB, b, x, Me, mdW=None, mdb=None):
    N = accA.shape[0]
    B = NB_B
    Din = x.shape[1]
    common_in = [
        pl.BlockSpec((B, 160), lambda i: (i, 0)),
        pl.BlockSpec((B, 160), lambda i: (i, 0)),
        pl.BlockSpec((B, 16), lambda i: (i, 0)),
        pl.BlockSpec((B, 16), lambda i: (i, 0)),
        pl.BlockSpec((1, HC), lambda i: (0, 0)),
        pl.BlockSpec((B, Din), lambda i: (i, 0)),
    ]
    if mdW is not None:
        in_specs = common_in + [
            pl.BlockSpec((Din, HC), lambda i: (0, 0)),
            pl.BlockSpec((1, HC), lambda i: (0, 0)),
            pl.BlockSpec((2, 16, 160), lambda i: (0, 0, 0)),
        ]
        args = (accA, accB, denA, denB, b.reshape(1, HC), x, mdW,
                mdb.reshape(1, HC), Me)
        body = _tcb_body_mlp
    else:
        in_specs = common_in + [pl.BlockSpec((2, 16, 160), lambda i: (0, 0, 0))]
        args = (accA, accB, denA, denB, b.reshape(1, HC), x, Me)
        body = _tcb_body_plain
    return pl.pallas_call(
        body,
        grid=(N // B,),
        in_specs=in_specs,
        out_specs=pl.BlockSpec((B, HC), lambda i: (i, 0)),
        out_shape=jax.ShapeDtypeStruct((N, HC), jnp.float32),
    )(*args)


def _pool_body(bf_r, x_r, s_r, cnt_r):
    i = pl.program_id(0)
    gids = lax.broadcasted_iota(jnp.int32, (1, G_), 1).astype(jnp.float32)
    mask = (bf_r[...] == gids).astype(jnp.float32)
    sp = lax.dot_general(mask, x_r[...], (((0,), (0,)), ((), ())),
                         preferred_element_type=jnp.float32)
    cp = lax.dot_general(mask, jnp.ones((mask.shape[0], 128), jnp.float32),
                         (((0,), (0,)), ((), ())),
                         preferred_element_type=jnp.float32)

    @pl.when(i == 0)
    def _():
        s_r[...] = sp
        cnt_r[...] = cp

    @pl.when(i != 0)
    def _():
        s_r[...] += sp
        cnt_r[...] += cp


def _pool(bf, x):
    N = x.shape[0]
    B = NB_B
    return pl.pallas_call(
        _pool_body,
        grid=(N // B,),
        in_specs=[
            pl.BlockSpec((B, 1), lambda i: (i, 0)),
            pl.BlockSpec((B, HC), lambda i: (i, 0)),
        ],
        out_specs=[
            pl.BlockSpec((G_, HC), lambda i: (0, 0)),
            pl.BlockSpec((G_, 128), lambda i: (0, 0)),
        ],
        out_shape=[
            jax.ShapeDtypeStruct((G_, HC), jnp.float32),
            jax.ShapeDtypeStruct((G_, 128), jnp.float32),
        ],
    )(bf, x)


def _head_body(s_r, cnt_r, w1_r, b1_r, w2_r, b2_r, lnw_r, lnb_r, out_r):
    cnt = jnp.maximum(cnt_r[:, 0:1], 1.0)
    xg = s_r[...] / cnt
    h1 = jnp.maximum(jnp.dot(xg, w1_r[...],
                             preferred_element_type=jnp.float32) + b1_r[...],
                     0.0)
    o = jnp.dot(h1, w2_r[...], preferred_element_type=jnp.float32) + b2_r[...]
    mu = jnp.mean(o, axis=-1, keepdims=True)
    var = jnp.mean((o - mu) ** 2, axis=-1, keepdims=True)
    out_r[...] = (o - mu) / jnp.sqrt(var + 1e-5) * lnw_r[...] + lnb_r[...]


def _headmlp(s, cnt, w1, b1, w2, b2, lnw, lnb):
    nhid = w1.shape[1]
    nout = w2.shape[1]
    return pl.pallas_call(
        _head_body,
        out_shape=jax.ShapeDtypeStruct((G_, nout), jnp.float32),
    )(s, cnt, w1, b1.reshape(1, nhid), w2, b2.reshape(1, nout),
      lnw.reshape(1, nout), lnb.reshape(1, nout))


@functools.lru_cache(maxsize=None)
def _sc_edge_build(N, Epad):
    from jax.experimental.pallas import tpu_sc as plsc

    NSUB = 16
    EPS = Epad // NSUB
    NCHUNK = EPS // E_K
    RPS = NP_PAD // NSUB
    WB_FULL = RPS // 8
    WB_TAIL = RPS - 8

    mesh = plsc.VectorSubcoreMesh(core_axis_name="c", subcore_axis_name="s")

    def body(h2_r, ald_r, src_r, dst_r, acc_out,
             acc_sh,
             src0, dst0, gidx0, rows0, ad0,
             src1, dst1, gidx1, rows1, ad1,
             src2, dst2, gidx2, rows2, ad2,
             w_v, tmp_v,
             gsem0, gsem1, gsem2, sr0, sr1, sr2):
        c = lax.axis_index("c")
        s = lax.axis_index("s")
        lane = lax.broadcasted_iota(jnp.int32, (16,), 0)
        zero = jnp.zeros((16,), jnp.float32)
        cb = jnp.zeros((16,), jnp.int32) + c
        is0 = cb == 0
        den_mask = jnp.where(is0, lane < 3, (lane >= 3) & (lane < 5))
        headv = [jnp.where(is0,
                           jnp.full((16,), v // 4, jnp.int32),
                           jnp.full((16,), (10 + v) // 4, jnp.int32))
                 for v in range(10)]
        cN16 = cb * N
        bufs = ((src0, dst0, gidx0, rows0, ad0, gsem0, sr0),
                (src1, dst1, gidx1, rows1, ad1, gsem1, sr1),
                (src2, dst2, gidx2, rows2, ad2, gsem2, sr2))
        ebase = s * EPS
        emax = Epad - E_K

        for r in range(8):
            for v in range(11):
                tmp_v[r, pl.ds(v * 16, 16)] = zero

        def zbody(k, carry):
            r0 = s * RPS + k * 8
            pltpu.sync_copy(tmp_v, acc_sh.at[pl.ds(r0, 8)])
            return carry

        lax.fori_loop(0, WB_FULL, zbody, 0)
        pltpu.sync_copy(tmp_v, acc_sh.at[pl.ds(s * RPS + WB_TAIL, 8)])
        plsc.subcore_barrier()

        def load_indices(B, cidx):
            src_v, dst_v, gidx_v = B[0], B[1], B[2]
            base = jnp.minimum(ebase + cidx * E_K, emax)
            pltpu.sync_copy(src_r.at[pl.ds(base, E_K)], src_v)
            pltpu.sync_copy(dst_r.at[pl.ds(base, E_K)], dst_v)
            for jj in range(E_K // 16):
                gidx_v[pl.ds(jj * 16, 16)] = src_v[pl.ds(jj * 16, 16)] + cN16

        def issue_gathers(B):
            pltpu.async_copy(h2_r.at[B[2]], B[3], B[5])
            pltpu.async_copy(ald_r.at[B[1]], B[4], B[5])

        def wait_gathers(B):
            pltpu.make_async_copy(h2_r.at[B[2]], B[3], B[5]).wait()
            pltpu.make_async_copy(ald_r.at[B[1]], B[4], B[5]).wait()

        def wait_scatter(B):
            pltpu.make_async_copy(B[3], acc_sh.at[B[1]], B[6]).wait()

        def compute_and_scatter(B):
            src_v, dst_v, gidx_v, rows_v, ad_v = B[:5]
            for j in range(E_K):
                e = rows_v[j, pl.ds(160, 16)] + ad_v[j]
                w = jnp.exp(jnp.maximum(e, 0.2 * e))
                w_v[pl.ds(j * 16, 16)] = w
                rows_v[j, pl.ds(160, 16)] = jnp.where(den_mask, w, 0.0)
            for j in range(E_K):
                for v in range(10):
                    splat = plsc.load_gather(w_v, [headv[v] + (j * 16)])
                    r = rows_v[j, pl.ds(v * 16, 16)]
                    rows_v[j, pl.ds(v * 16, 16)] = r * splat
            pltpu.async_copy(rows_v, acc_sh.at[dst_v], B[6], add=True)

        def step_rest(cidx, Bc, Bn):
            load_indices(Bn, cidx + 1)
            issue_gathers(Bn)
            wait_gathers(Bc)
            compute_and_scatter(Bc)

        load_indices(bufs[0], jnp.int32(0))
        issue_gathers(bufs[0])

        def ebody(t, carry):
            c0 = 3 * t

            @pl.when(t > 0)
            def _():
                wait_scatter(bufs[1])

            step_rest(c0, bufs[0], bufs[1])

            @pl.when(t > 0)
            def _():
                wait_scatter(bufs[2])

            step_rest(c0 + 1, bufs[1], bufs[2])
            wait_scatter(bufs[0])
            step_rest(c0 + 2, bufs[2], bufs[0])
            return carry

        lax.fori_loop(0, NCHUNK // 3, ebody, 0)

        wait_gathers(bufs[0])
        wait_scatter(bufs[1])
        wait_scatter(bufs[2])
        plsc.subcore_barrier()

        def wbody(k, carry):
            r0 = s * RPS + k * 8
            pltpu.sync_copy(acc_sh.at[pl.ds(r0, 8)], tmp_v)
            pltpu.sync_copy(tmp_v, acc_out.at[pl.ds(c * NP_PAD + r0, 8)])
            return carry

        lax.fori_loop(0, WB_FULL, wbody, 0)
        r0 = s * RPS + WB_TAIL
        pltpu.sync_copy(acc_sh.at[pl.ds(r0, 8)], tmp_v)
        pltpu.sync_copy(tmp_v, acc_out.at[pl.ds(c * NP_PAD + r0, 8)])

    return pl.kernel(
        body,
        out_type=jax.ShapeDtypeStruct((2 * NP_PAD, 176), jnp.float32),
        mesh=mesh,
        compiler_params=pltpu.CompilerParams(needs_layout_passes=False,
                                             use_tc_tiling_on_sc=False),
        scratch_types=(
            [pltpu.VMEM_SHARED((NP_PAD, 176), jnp.float32)]
            + [pltpu.VMEM((E_K,), jnp.int32),
               pltpu.VMEM((E_K,), jnp.int32),
               pltpu.VMEM((E_K,), jnp.int32),
               pltpu.VMEM((E_K, 176), jnp.float32),
               pltpu.VMEM((E_K, 16), jnp.float32)] * 3
            + [pltpu.VMEM((E_K * 16,), jnp.float32),
               pltpu.VMEM((8, 176), jnp.float32)]
            + [pltpu.SemaphoreType.DMA] * 6
        ),
    )


def _edge_sc(h2, aldp, srcp, dstp):
    N = aldp.shape[0]
    Epad = srcp.shape[0]
    h2f = h2.reshape(2 * N, 176)
    acc = _sc_edge_build(N, Epad)(h2f, aldp, srcp, dstp)
    acc = acc.reshape(2, NP_PAD, 176)
    return (acc[0, :N, :160], acc[1, :N, :160],
            acc[0, :N, 160:], acc[1, :N, 160:])


def kernel(x, edge_index, batch, c1_W, c1_as, c1_ad, c1_b, c2_W, c2_as, c2_ad,
           c2_b, c3_W, c3_as, c3_ad, c3_b, c4_W, c4_as, c4_ad, c4_b, c5_W,
           c5_as, c5_ad, c5_b, md1_W, md1_b, md2_W, md2_b, mh1_W, mh1_b,
           mh2_W, mh2_b, ln_w, ln_b):
    N = x.shape[0]
    loop = jnp.arange(N, dtype=edge_index.dtype)
    E_tot = edge_index.shape[1] + N
    Egrain = 16 * 3 * E_K
    Epad = ((E_tot + Egrain - 1) // Egrain) * Egrain
    padn = Epad - E_tot
    srcp = jnp.concatenate(
        [edge_index[0], loop, jnp.zeros((padn,), edge_index.dtype)])
    dstp = jnp.concatenate(
        [edge_index[1], loop,
         jnp.full((padn,), NP_PAD - 1, edge_index.dtype)])
    Mh, Me = _head_maps()

    layers = [
        (c1_W, c1_as, c1_ad, c1_b, md1_W, md1_b),
        (c2_W, c2_as, c2_ad, c2_b, md2_W, md2_b),
        (c3_W, c3_as, c3_ad, c3_b, None, None),
        (c4_W, c4_as, c4_ad, c4_b, None, None),
        (c5_W, c5_as, c5_ad, c5_b, None, None),
    ]

    xcur = x
    for (W, a_s, a_d, b, mdW, mdb) in layers:
        asv = a_s.reshape(1, HC)
        adv = a_d.reshape(1, HC)
        h2, aldp = _tca(xcur, W, asv, adv, Mh)
        accA, accB, denA, denB = _edge_sc(h2, aldp, srcp, dstp)
        xcur = _tcb(accA, accB, denA, denB, b, xcur, Me, mdW, mdb)

    bf = batch.astype(jnp.float32).reshape(N, 1)
    s, cnt = _pool(bf, xcur)
    return _headmlp(s, cnt, mh1_W, mh1_b, mh2_W, mh2_b, ln_w, ln_b)


# speedup vs baseline: 26.8943x; 1.5401x over previous
import functools

import jax
import jax.numpy as jnp
import numpy as np
from jax import lax
from jax.experimental import pallas as pl
from jax.experimental.pallas import tpu as pltpu

H_ = 5
C_ = 64
HC = 320
G_ = 64
NB_B = 1000
NP_PAD = 10016
E_K = 16


def _head_maps():
    Mh = np.zeros((320, 16), np.float32)
    Me = np.zeros((2, 16, 160), np.float32)
    for c in range(320):
        h = c // 64
        Mh[c, h] = 1.0
        Me[c // 160, h, c % 160] = 1.0
    return jnp.asarray(Mh), jnp.asarray(Me)


def _tca_body(x_ref, W_ref, asv_ref, adv_ref, M_ref, h2_ref, ald_ref):
    hb = jnp.dot(x_ref[...], W_ref[...], preferred_element_type=jnp.float32)
    als = jnp.dot(hb * asv_ref[...], M_ref[...],
                  preferred_element_type=jnp.float32)
    ald_ref[...] = jnp.dot(hb * adv_ref[...], M_ref[...],
                           preferred_element_type=jnp.float32)
    blk0 = jnp.concatenate([hb[:, :160], als], axis=-1)
    blk1 = jnp.concatenate([hb[:, 160:], als], axis=-1)
    h2_ref[...] = jnp.stack([blk0, blk1], axis=0)


def _tca(x, W, asv, adv, Mh):
    N, Din = x.shape
    B = NB_B
    return pl.pallas_call(
        _tca_body,
        grid=(N // B,),
        in_specs=[
            pl.BlockSpec((B, Din), lambda i: (i, 0)),
            pl.BlockSpec((Din, HC), lambda i: (0, 0)),
            pl.BlockSpec((1, HC), lambda i: (0, 0)),
            pl.BlockSpec((1, HC), lambda i: (0, 0)),
            pl.BlockSpec((HC, 16), lambda i: (0, 0)),
        ],
        out_specs=[
            pl.BlockSpec((2, B, 176), lambda i: (0, i, 0)),
            pl.BlockSpec((B, 16), lambda i: (i, 0)),
        ],
        out_shape=[
            jax.ShapeDtypeStruct((2, N, 176), jnp.float32),
            jax.ShapeDtypeStruct((N, 16), jnp.float32),
        ],
    )(x, W, asv, adv, Mh)


def _tcb_body_mlp(accA_r, accB_r, denA_r, denB_r, b_r, x_r, mdW_r, mdb_r,
                  Me_r, out_r):
    den16 = denA_r[...] + denB_r[...]
    dA = jnp.dot(den16, Me_r[0], preferred_element_type=jnp.float32)
    dB = jnp.dot(den16, Me_r[1], preferred_element_type=jnp.float32)
    msg = jnp.concatenate([accA_r[...] / dA, accB_r[...] / dB], axis=-1)
    x0 = jnp.maximum(msg + b_r[...], 0.0)
    out_r[...] = (jnp.dot(x_r[...], mdW_r[...],
                          preferred_element_type=jnp.float32)
                  + mdb_r[...] + x0)


def _tcb_body_plain(accA_r, accB_r, denA_r, denB_r, b_r, x_r, Me_r, out_r):
    den16 = denA_r[...] + denB_r[...]
    dA = jnp.dot(den16, Me_r[0], preferred_element_type=jnp.float32)
    dB = jnp.dot(den16, Me_r[1], preferred_element_type=jnp.float32)
    msg = jnp.concatenate([accA_r[...] / dA, accB_r[...] / dB], axis=-1)
    x0 = jnp.maximum(msg + b_r[...], 0.0)
    out_r[...] = x_r[...] + x0


def _tcb(accA, accB, denA, denB, b, x, Me, mdW=None, mdb=None):
    N = accA.shape[0]
    B = NB_B
    Din = x.shape[1]
    common_in = [
        pl.BlockSpec((B, 160), lambda i: (i, 0)),
        pl.BlockSpec((B, 160), lambda i: (i, 0)),
        pl.BlockSpec((B, 16), lambda i: (i, 0)),
        pl.BlockSpec((B, 16), lambda i: (i, 0)),
        pl.BlockSpec((1, HC), lambda i: (0, 0)),
        pl.BlockSpec((B, Din), lambda i: (i, 0)),
    ]
    if mdW is not None:
        in_specs = common_in + [
            pl.BlockSpec((Din, HC), lambda i: (0, 0)),
            pl.BlockSpec((1, HC), lambda i: (0, 0)),
            pl.BlockSpec((2, 16, 160), lambda i: (0, 0, 0)),
        ]
        args = (accA, accB, denA, denB, b.reshape(1, HC), x, mdW,
                mdb.reshape(1, HC), Me)
        body = _tcb_body_mlp
    else:
        in_specs = common_in + [pl.BlockSpec((2, 16, 160), lambda i: (0, 0, 0))]
        args = (accA, accB, denA, denB, b.reshape(1, HC), x, Me)
        body = _tcb_body_plain
    return pl.pallas_call(
        body,
        grid=(N // B,),
        in_specs=in_specs,
        out_specs=pl.BlockSpec((B, HC), lambda i: (i, 0)),
        out_shape=jax.ShapeDtypeStruct((N, HC), jnp.float32),
    )(*args)


def _pool_body(bf_r, x_r, s_r, cnt_r):
    i = pl.program_id(0)
    gids = lax.broadcasted_iota(jnp.int32, (1, G_), 1).astype(jnp.float32)
    mask = (bf_r[...] == gids).astype(jnp.float32)
    sp = lax.dot_general(mask, x_r[...], (((0,), (0,)), ((), ())),
                         preferred_element_type=jnp.float32)
    cp = lax.dot_general(mask, jnp.ones((mask.shape[0], 128), jnp.float32),
                         (((0,), (0,)), ((), ())),
                         preferred_element_type=jnp.float32)

    @pl.when(i == 0)
    def _():
        s_r[...] = sp
        cnt_r[...] = cp

    @pl.when(i != 0)
    def _():
        s_r[...] += sp
        cnt_r[...] += cp


def _pool(bf, x):
    N = x.shape[0]
    B = NB_B
    return pl.pallas_call(
        _pool_body,
        grid=(N // B,),
        in_specs=[
            pl.BlockSpec((B, 1), lambda i: (i, 0)),
            pl.BlockSpec((B, HC), lambda i: (i, 0)),
        ],
        out_specs=[
            pl.BlockSpec((G_, HC), lambda i: (0, 0)),
            pl.BlockSpec((G_, 128), lambda i: (0, 0)),
        ],
        out_shape=[
            jax.ShapeDtypeStruct((G_, HC), jnp.float32),
            jax.ShapeDtypeStruct((G_, 128), jnp.float32),
        ],
    )(bf, x)


def _head_body(s_r, cnt_r, w1_r, b1_r, w2_r, b2_r, lnw_r, lnb_r, out_r):
    cnt = jnp.maximum(cnt_r[:, 0:1], 1.0)
    xg = s_r[...] / cnt
    h1 = jnp.maximum(jnp.dot(xg, w1_r[...],
                             preferred_element_type=jnp.float32) + b1_r[...],
                     0.0)
    o = jnp.dot(h1, w2_r[...], preferred_element_type=jnp.float32) + b2_r[...]
    mu = jnp.mean(o, axis=-1, keepdims=True)
    var = jnp.mean((o - mu) ** 2, axis=-1, keepdims=True)
    out_r[...] = (o - mu) / jnp.sqrt(var + 1e-5) * lnw_r[...] + lnb_r[...]


def _headmlp(s, cnt, w1, b1, w2, b2, lnw, lnb):
    nhid = w1.shape[1]
    nout = w2.shape[1]
    return pl.pallas_call(
        _head_body,
        out_shape=jax.ShapeDtypeStruct((G_, nout), jnp.float32),
    )(s, cnt, w1, b1.reshape(1, nhid), w2, b2.reshape(1, nout),
      lnw.reshape(1, nout), lnb.reshape(1, nout))


@functools.lru_cache(maxsize=None)
def _sc_edge_build(N, Epad):
    from jax.experimental.pallas import tpu_sc as plsc

    NSUB = 16
    EPS = Epad // NSUB
    NCHUNK = EPS // E_K
    RPS = NP_PAD // NSUB
    WB_FULL = RPS // 8
    WB_TAIL = RPS - 8

    mesh = plsc.VectorSubcoreMesh(core_axis_name="c", subcore_axis_name="s")

    def body(h2_r, ald_r, src_r, dst_r, acc_out,
             acc_sh,
             src0, dst0, gidx0, rows0, ad0, sdst0,
             src1, dst1, gidx1, rows1, ad1, sdst1,
             src2, dst2, gidx2, rows2, ad2, sdst2,
             w_v, tmp_v,
             gsem0, gsem1, gsem2, sr0, sr1, sr2, is0_, is1_, is2_):
        c = lax.axis_index("c")
        s = lax.axis_index("s")
        lane = lax.broadcasted_iota(jnp.int32, (16,), 0)
        zero = jnp.zeros((16,), jnp.float32)
        cb = jnp.zeros((16,), jnp.int32) + c
        is0 = cb == 0
        den_mask = jnp.where(is0, lane < 3, (lane >= 3) & (lane < 5))
        headv = [jnp.where(is0,
                           jnp.full((16,), v // 4, jnp.int32),
                           jnp.full((16,), (10 + v) // 4, jnp.int32))
                 for v in range(10)]
        cN16 = cb * N
        bufs = ((src0, dst0, gidx0, rows0, ad0, sdst0, gsem0, sr0, is0_),
                (src1, dst1, gidx1, rows1, ad1, sdst1, gsem1, sr1, is1_),
                (src2, dst2, gidx2, rows2, ad2, sdst2, gsem2, sr2, is2_))
        ebase = s * EPS
        emax = Epad - E_K

        for r in range(8):
            for v in range(11):
                tmp_v[r, pl.ds(v * 16, 16)] = zero

        def zbody(k, carry):
            r0 = s * RPS + k * 8
            pltpu.sync_copy(tmp_v, acc_sh.at[pl.ds(r0, 8)])
            return carry

        lax.fori_loop(0, WB_FULL, zbody, 0)
        pltpu.sync_copy(tmp_v, acc_sh.at[pl.ds(s * RPS + WB_TAIL, 8)])
        plsc.subcore_barrier()

        def idx_base(cidx):
            return jnp.minimum(ebase + cidx * E_K, emax)

        def issue_idx(B, cidx):
            base = idx_base(cidx)
            pltpu.async_copy(src_r.at[pl.ds(base, E_K)], B[0], B[8])
            pltpu.async_copy(dst_r.at[pl.ds(base, E_K)], B[1], B[8])

        def wait_idx(B, cidx):
            base = idx_base(cidx)
            pltpu.make_async_copy(src_r.at[pl.ds(base, E_K)], B[0],
                                  B[8]).wait()
            pltpu.make_async_copy(dst_r.at[pl.ds(base, E_K)], B[1],
                                  B[8]).wait()

        def issue_gathers(B):
            for jj in range(E_K // 16):
                B[2][pl.ds(jj * 16, 16)] = B[0][pl.ds(jj * 16, 16)] + cN16
            pltpu.async_copy(h2_r.at[B[2]], B[3], B[6])
            pltpu.async_copy(ald_r.at[B[1]], B[4], B[6])

        def wait_gathers(B):
            pltpu.make_async_copy(h2_r.at[B[2]], B[3], B[6]).wait()
            pltpu.make_async_copy(ald_r.at[B[1]], B[4], B[6]).wait()

        def wait_scatter(B):
            pltpu.make_async_copy(B[3], acc_sh.at[B[5]], B[7]).wait()

        def compute_and_scatter(B):
            src_v, dst_v, gidx_v, rows_v, ad_v, sdst_v = B[:6]
            for j in range(E_K):
                e = rows_v[j, pl.ds(160, 16)] + ad_v[j]
                w = jnp.exp(jnp.maximum(e, 0.2 * e))
                w_v[pl.ds(j * 16, 16)] = w
                rows_v[j, pl.ds(160, 16)] = jnp.where(den_mask, w, 0.0)
            for jj in range(E_K // 16):
                sdst_v[pl.ds(jj * 16, 16)] = dst_v[pl.ds(jj * 16, 16)]
            for j in range(E_K):
                for v in range(10):
                    splat = plsc.load_gather(w_v, [headv[v] + (j * 16)])
                    r = rows_v[j, pl.ds(v * 16, 16)]
                    rows_v[j, pl.ds(v * 16, 16)] = r * splat
            pltpu.async_copy(rows_v, acc_sh.at[sdst_v], B[7], add=True)

        def step_rest(cidx, B, Bn, Bf):
            wait_idx(Bn, cidx + 1)
            issue_gathers(Bn)
            issue_idx(Bf, cidx + 2)
            wait_gathers(B)
            compute_and_scatter(B)

        base0 = idx_base(jnp.int32(0))
        pltpu.sync_copy(src_r.at[pl.ds(base0, E_K)], bufs[0][0])
        pltpu.sync_copy(dst_r.at[pl.ds(base0, E_K)], bufs[0][1])
        issue_gathers(bufs[0])
        issue_idx(bufs[1], jnp.int32(1))

        def ebody(t, carry):
            c0 = 3 * t

            @pl.when(t > 0)
            def _():
                wait_scatter(bufs[1])

            step_rest(c0, bufs[0], bufs[1], bufs[2])

            @pl.when(t > 0)
            def _():
                wait_scatter(bufs[2])

            step_rest(c0 + 1, bufs[1], bufs[2], bufs[0])
            wait_scatter(bufs[0])
            step_rest(c0 + 2, bufs[2], bufs[0], bufs[1])
            return carry

        lax.fori_loop(0, NCHUNK // 3, ebody, 0)

        wait_gathers(bufs[0])
        wait_idx(bufs[1], jnp.int32(NCHUNK + 1))
        wait_scatter(bufs[1])
        wait_scatter(bufs[2])
        plsc.subcore_barrier()

        def wbody(k, carry):
            r0 = s * RPS + k * 8
            pltpu.sync_copy(acc_sh.at[pl.ds(r0, 8)], tmp_v)
            pltpu.sync_copy(tmp_v, acc_out.at[pl.ds(c * NP_PAD + r0, 8)])
            return carry

        lax.fori_loop(0, WB_FULL, wbody, 0)
        r0 = s * RPS + WB_TAIL
        pltpu.sync_copy(acc_sh.at[pl.ds(r0, 8)], tmp_v)
        pltpu.sync_copy(tmp_v, acc_out.at[pl.ds(c * NP_PAD + r0, 8)])

    return pl.kernel(
        body,
        out_type=jax.ShapeDtypeStruct((2 * NP_PAD, 176), jnp.float32),
        mesh=mesh,
        compiler_params=pltpu.CompilerParams(needs_layout_passes=False,
                                             use_tc_tiling_on_sc=False),
        scratch_types=(
            [pltpu.VMEM_SHARED((NP_PAD, 176), jnp.float32)]
            + [pltpu.VMEM((E_K,), jnp.int32),
               pltpu.VMEM((E_K,), jnp.int32),
               pltpu.VMEM((E_K,), jnp.int32),
               pltpu.VMEM((E_K, 176), jnp.float32),
               pltpu.VMEM((E_K, 16), jnp.float32),
               pltpu.VMEM((E_K,), jnp.int32)] * 3
            + [pltpu.VMEM((E_K * 16,), jnp.float32),
               pltpu.VMEM((8, 176), jnp.float32)]
            + [pltpu.SemaphoreType.DMA] * 9
        ),
    )


def _edge_sc(h2, aldp, srcp, dstp):
    N = aldp.shape[0]
    Epad = srcp.shape[0]
    h2f = h2.reshape(2 * N, 176)
    acc = _sc_edge_build(N, Epad)(h2f, aldp, srcp, dstp)
    acc = acc.reshape(2, NP_PAD, 176)
    return (acc[0, :N, :160], acc[1, :N, :160],
            acc[0, :N, 160:], acc[1, :N, 160:])


def kernel(x, edge_index, batch, c1_W, c1_as, c1_ad, c1_b, c2_W, c2_as, c2_ad,
           c2_b, c3_W, c3_as, c3_ad, c3_b, c4_W, c4_as, c4_ad, c4_b, c5_W,
           c5_as, c5_ad, c5_b, md1_W, md1_b, md2_W, md2_b, mh1_W, mh1_b,
           mh2_W, mh2_b, ln_w, ln_b):
    N = x.shape[0]
    loop = jnp.arange(N, dtype=edge_index.dtype)
    E_tot = edge_index.shape[1] + N
    Egrain = 16 * 3 * E_K
    Epad = ((E_tot + Egrain - 1) // Egrain) * Egrain
    padn = Epad - E_tot
    srcp = jnp.concatenate(
        [edge_index[0], loop, jnp.zeros((padn,), edge_index.dtype)])
    dstp = jnp.concatenate(
        [edge_index[1], loop,
         jnp.full((padn,), NP_PAD - 1, edge_index.dtype)])
    Mh, Me = _head_maps()

    layers = [
        (c1_W, c1_as, c1_ad, c1_b, md1_W, md1_b),
        (c2_W, c2_as, c2_ad, c2_b, md2_W, md2_b),
        (c3_W, c3_as, c3_ad, c3_b, None, None),
        (c4_W, c4_as, c4_ad, c4_b, None, None),
        (c5_W, c5_as, c5_ad, c5_b, None, None),
    ]

    xcur = x
    for (W, a_s, a_d, b, mdW, mdb) in layers:
        asv = a_s.reshape(1, HC)
        adv = a_d.reshape(1, HC)
        h2, aldp = _tca(xcur, W, asv, adv, Mh)
        accA, accB, denA, denB = _edge_sc(h2, aldp, srcp, dstp)
        xcur = _tcb(accA, accB, denA, denB, b, xcur, Me, mdW, mdb)

    bf = batch.astype(jnp.float32).reshape(N, 1)
    s, cnt = _pool(bf, xcur)
    return _headmlp(s, cnt, mh1_W, mh1_b, mh2_W, mh2_b, ln_w, ln_b)


# speedup vs baseline: 27.7917x; 1.0334x over previous
import functools

import jax
import jax.numpy as jnp
import numpy as np
from jax import lax
from jax.experimental import pallas as pl
from jax.experimental.pallas import tpu as pltpu

H_ = 5
C_ = 64
HC = 320
G_ = 64
NB_B = 1000
NP_PAD = 10016
E_K = 32


def _head_maps():
    Mh = np.zeros((320, 16), np.float32)
    Me = np.zeros((2, 16, 160), np.float32)
    for c in range(320):
        h = c // 64
        Mh[c, h] = 1.0
        Me[c // 160, h, c % 160] = 1.0
    return jnp.asarray(Mh), jnp.asarray(Me)


def _tca_body(x_ref, W_ref, asv_ref, adv_ref, M_ref, h2_ref, ald_ref):
    hb = jnp.dot(x_ref[...], W_ref[...], preferred_element_type=jnp.float32)
    als = jnp.dot(hb * asv_ref[...], M_ref[...],
                  preferred_element_type=jnp.float32)
    ald_ref[...] = jnp.dot(hb * adv_ref[...], M_ref[...],
                           preferred_element_type=jnp.float32)
    blk0 = jnp.concatenate([hb[:, :160], als], axis=-1)
    blk1 = jnp.concatenate([hb[:, 160:], als], axis=-1)
    h2_ref[...] = jnp.stack([blk0, blk1], axis=0)


def _tca(x, W, asv, adv, Mh):
    N, Din = x.shape
    B = NB_B
    return pl.pallas_call(
        _tca_body,
        grid=(N // B,),
        in_specs=[
            pl.BlockSpec((B, Din), lambda i: (i, 0)),
            pl.BlockSpec((Din, HC), lambda i: (0, 0)),
            pl.BlockSpec((1, HC), lambda i: (0, 0)),
            pl.BlockSpec((1, HC), lambda i: (0, 0)),
            pl.BlockSpec((HC, 16), lambda i: (0, 0)),
        ],
        out_specs=[
            pl.BlockSpec((2, B, 176), lambda i: (0, i, 0)),
            pl.BlockSpec((B, 16), lambda i: (i, 0)),
        ],
        out_shape=[
            jax.ShapeDtypeStruct((2, N, 176), jnp.float32),
            jax.ShapeDtypeStruct((N, 16), jnp.float32),
        ],
    )(x, W, asv, adv, Mh)


def _tcb_body_mlp(accA_r, accB_r, denA_r, denB_r, b_r, x_r, mdW_r, mdb_r,
                  Me_r, out_r):
    den16 = denA_r[...] + denB_r[...]
    dA = jnp.dot(den16, Me_r[0], preferred_element_type=jnp.float32)
    dB = jnp.dot(den16, Me_r[1], preferred_element_type=jnp.float32)
    msg = jnp.concatenate([accA_r[...] / dA, accB_r[...] / dB], axis=-1)
    x0 = jnp.maximum(msg + b_r[...], 0.0)
    out_r[...] = (jnp.dot(x_r[...], mdW_r[...],
                          preferred_element_type=jnp.float32)
                  + mdb_r[...] + x0)


def _tcb_body_plain(accA_r, accB_r, denA_r, denB_r, b_r, x_r, Me_r, out_r):
    den16 = denA_r[...] + denB_r[...]
    dA = jnp.dot(den16, Me_r[0], preferred_element_type=jnp.float32)
    dB = jnp.dot(den16, Me_r[1], preferred_element_type=jnp.float32)
    msg = jnp.concatenate([accA_r[...] / dA, accB_r[...] / dB], axis=-1)
    x0 = jnp.maximum(msg + b_r[...], 0.0)
    out_r[...] = x_r[...] + x0


def _tcb(accA, accB, denA, denB, b, x, Me, mdW=None, mdb=None):
    N = accA.shape[0]
    B = NB_B
    Din = x.shape[1]
    common_in = [
        pl.BlockSpec((B, 160), lambda i: (i, 0)),
        pl.BlockSpec((B, 160), lambda i: (i, 0)),
        pl.BlockSpec((B, 16), lambda i: (i, 0)),
        pl.BlockSpec((B, 16), lambda i: (i, 0)),
        pl.BlockSpec((1, HC), lambda i: (0, 0)),
        pl.BlockSpec((B, Din), lambda i: (i, 0)),
    ]
    if mdW is not None:
        in_specs = common_in + [
            pl.BlockSpec((Din, HC), lambda i: (0, 0)),
            pl.BlockSpec((1, HC), lambda i: (0, 0)),
            pl.BlockSpec((2, 16, 160), lambda i: (0, 0, 0)),
        ]
        args = (accA, accB, denA, denB, b.reshape(1, HC), x, mdW,
                mdb.reshape(1, HC), Me)
        body = _tcb_body_mlp
    else:
        in_specs = common_in + [pl.BlockSpec((2, 16, 160), lambda i: (0, 0, 0))]
        args = (accA, accB, denA, denB, b.reshape(1, HC), x, Me)
        body = _tcb_body_plain
    return pl.pallas_call(
        body,
        grid=(N // B,),
        in_specs=in_specs,
        out_specs=pl.BlockSpec((B, HC), lambda i: (i, 0)),
        out_shape=jax.ShapeDtypeStruct((N, HC), jnp.float32),
    )(*args)


def _pool_body(bf_r, x_r, s_r, cnt_r):
    i = pl.program_id(0)
    gids = lax.broadcasted_iota(jnp.int32, (1, G_), 1).astype(jnp.float32)
    mask = (bf_r[...] == gids).astype(jnp.float32)
    sp = lax.dot_general(mask, x_r[...], (((0,), (0,)), ((), ())),
                         preferred_element_type=jnp.float32)
    cp = lax.dot_general(mask, jnp.ones((mask.shape[0], 128), jnp.float32),
                         (((0,), (0,)), ((), ())),
                         preferred_element_type=jnp.float32)

    @pl.when(i == 0)
    def _():
        s_r[...] = sp
        cnt_r[...] = cp

    @pl.when(i != 0)
    def _():
        s_r[...] += sp
        cnt_r[...] += cp


def _pool(bf, x):
    N = x.shape[0]
    B = NB_B
    return pl.pallas_call(
        _pool_body,
        grid=(N // B,),
        in_specs=[
            pl.BlockSpec((B, 1), lambda i: (i, 0)),
            pl.BlockSpec((B, HC), lambda i: (i, 0)),
        ],
        out_specs=[
            pl.BlockSpec((G_, HC), lambda i: (0, 0)),
            pl.BlockSpec((G_, 128), lambda i: (0, 0)),
        ],
        out_shape=[
            jax.ShapeDtypeStruct((G_, HC), jnp.float32),
            jax.ShapeDtypeStruct((G_, 128), jnp.float32),
        ],
    )(bf, x)


def _head_body(s_r, cnt_r, w1_r, b1_r, w2_r, b2_r, lnw_r, lnb_r, out_r):
    cnt = jnp.maximum(cnt_r[:, 0:1], 1.0)
    xg = s_r[...] / cnt
    h1 = jnp.maximum(jnp.dot(xg, w1_r[...],
                             preferred_element_type=jnp.float32) + b1_r[...],
                     0.0)
    o = jnp.dot(h1, w2_r[...], preferred_element_type=jnp.float32) + b2_r[...]
    mu = jnp.mean(o, axis=-1, keepdims=True)
    var = jnp.mean((o - mu) ** 2, axis=-1, keepdims=True)
    out_r[...] = (o - mu) / jnp.sqrt(var + 1e-5) * lnw_r[...] + lnb_r[...]


def _headmlp(s, cnt, w1, b1, w2, b2, lnw, lnb):
    nhid = w1.shape[1]
    nout = w2.shape[1]
    return pl.pallas_call(
        _head_body,
        out_shape=jax.ShapeDtypeStruct((G_, nout), jnp.float32),
    )(s, cnt, w1, b1.reshape(1, nhid), w2, b2.reshape(1, nout),
      lnw.reshape(1, nout), lnb.reshape(1, nout))


@functools.lru_cache(maxsize=None)
def _sc_edge_build(N, Epad):
    from jax.experimental.pallas import tpu_sc as plsc

    NSUB = 16
    EPS = Epad // NSUB
    NCHUNK = EPS // E_K
    RPS = NP_PAD // NSUB
    WB_FULL = RPS // 8
    WB_TAIL = RPS - 8

    mesh = plsc.VectorSubcoreMesh(core_axis_name="c", subcore_axis_name="s")

    def body(h2_r, ald_r, src_r, dst_r, acc_out,
             acc_sh,
             src0, dst0, gidx0, rows0, ad0, sdst0,
             src1, dst1, gidx1, rows1, ad1, sdst1,
             src2, dst2, gidx2, rows2, ad2, sdst2,
             w_v, tmp_v,
             gsem0, gsem1, gsem2, sr0, sr1, sr2, is0_, is1_, is2_):
        c = lax.axis_index("c")
        s = lax.axis_index("s")
        lane = lax.broadcasted_iota(jnp.int32, (16,), 0)
        zero = jnp.zeros((16,), jnp.float32)
        cb = jnp.zeros((16,), jnp.int32) + c
        is0 = cb == 0
        den_mask = jnp.where(is0, lane < 3, (lane >= 3) & (lane < 5))
        headv = [jnp.where(is0,
                           jnp.full((16,), v // 4, jnp.int32),
                           jnp.full((16,), (10 + v) // 4, jnp.int32))
                 for v in range(10)]
        cN16 = cb * N
        bufs = ((src0, dst0, gidx0, rows0, ad0, sdst0, gsem0, sr0, is0_),
                (src1, dst1, gidx1, rows1, ad1, sdst1, gsem1, sr1, is1_),
                (src2, dst2, gidx2, rows2, ad2, sdst2, gsem2, sr2, is2_))
        ebase = s * EPS
        emax = Epad - E_K

        for r in range(8):
            for v in range(11):
                tmp_v[r, pl.ds(v * 16, 16)] = zero

        def zbody(k, carry):
            r0 = s * RPS + k * 8
            pltpu.sync_copy(tmp_v, acc_sh.at[pl.ds(r0, 8)])
            return carry

        lax.fori_loop(0, WB_FULL, zbody, 0)
        pltpu.sync_copy(tmp_v, acc_sh.at[pl.ds(s * RPS + WB_TAIL, 8)])
        plsc.subcore_barrier()

        def idx_base(cidx):
            return jnp.minimum(ebase + cidx * E_K, emax)

        def issue_idx(B, cidx):
            base = idx_base(cidx)
            pltpu.async_copy(src_r.at[pl.ds(base, E_K)], B[0], B[8])
            pltpu.async_copy(dst_r.at[pl.ds(base, E_K)], B[1], B[8])

        def wait_idx(B, cidx):
            base = idx_base(cidx)
            pltpu.make_async_copy(src_r.at[pl.ds(base, E_K)], B[0],
                                  B[8]).wait()
            pltpu.make_async_copy(dst_r.at[pl.ds(base, E_K)], B[1],
                                  B[8]).wait()

        def issue_gathers(B):
            for jj in range(E_K // 16):
                B[2][pl.ds(jj * 16, 16)] = B[0][pl.ds(jj * 16, 16)] + cN16
            pltpu.async_copy(h2_r.at[B[2]], B[3], B[6])
            pltpu.async_copy(ald_r.at[B[1]], B[4], B[6])

        def wait_gathers(B):
            pltpu.make_async_copy(h2_r.at[B[2]], B[3], B[6]).wait()
            pltpu.make_async_copy(ald_r.at[B[1]], B[4], B[6]).wait()

        def wait_scatter(B):
            pltpu.make_async_copy(B[3], acc_sh.at[B[5]], B[7]).wait()

        def compute_and_scatter(B):
            src_v, dst_v, gidx_v, rows_v, ad_v, sdst_v = B[:6]

            def wbody2(j, carry):
                e = rows_v[j, pl.ds(160, 16)] + ad_v[j]
                w = jnp.exp(jnp.maximum(e, 0.2 * e))
                w_v[pl.ds(j * 16, 16)] = w
                rows_v[j, pl.ds(160, 16)] = jnp.where(den_mask, w, 0.0)
                return carry

            lax.fori_loop(0, E_K, wbody2, 0)
            for jj in range(E_K // 16):
                sdst_v[pl.ds(jj * 16, 16)] = dst_v[pl.ds(jj * 16, 16)]

            def sbody(j, carry):
                jb = j * 16
                for v in range(10):
                    splat = plsc.load_gather(w_v, [headv[v] + jb])
                    r = rows_v[j, pl.ds(v * 16, 16)]
                    rows_v[j, pl.ds(v * 16, 16)] = r * splat
                return carry

            lax.fori_loop(0, E_K, sbody, 0)
            pltpu.async_copy(rows_v, acc_sh.at[sdst_v], B[7], add=True)

        def step_rest(cidx, B, Bn, Bf):
            wait_idx(Bn, cidx + 1)
            issue_gathers(Bn)
            issue_idx(Bf, cidx + 2)
            wait_gathers(B)
            compute_and_scatter(B)

        base0 = idx_base(jnp.int32(0))
        pltpu.sync_copy(src_r.at[pl.ds(base0, E_K)], bufs[0][0])
        pltpu.sync_copy(dst_r.at[pl.ds(base0, E_K)], bufs[0][1])
        issue_gathers(bufs[0])
        issue_idx(bufs[1], jnp.int32(1))

        def ebody(t, carry):
            c0 = 3 * t

            @pl.when(t > 0)
            def _():
                wait_scatter(bufs[1])

            step_rest(c0, bufs[0], bufs[1], bufs[2])

            @pl.when(t > 0)
            def _():
                wait_scatter(bufs[2])

            step_rest(c0 + 1, bufs[1], bufs[2], bufs[0])
            wait_scatter(bufs[0])
            step_rest(c0 + 2, bufs[2], bufs[0], bufs[1])
            return carry

        lax.fori_loop(0, NCHUNK // 3, ebody, 0)

        wait_gathers(bufs[0])
        wait_idx(bufs[1], jnp.int32(NCHUNK + 1))
        wait_scatter(bufs[1])
        wait_scatter(bufs[2])
        plsc.subcore_barrier()

        def wbody(k, carry):
            r0 = s * RPS + k * 8
            pltpu.sync_copy(acc_sh.at[pl.ds(r0, 8)], tmp_v)
            pltpu.sync_copy(tmp_v, acc_out.at[pl.ds(c * NP_PAD + r0, 8)])
            return carry

        lax.fori_loop(0, WB_FULL, wbody, 0)
        r0 = s * RPS + WB_TAIL
        pltpu.sync_copy(acc_sh.at[pl.ds(r0, 8)], tmp_v)
        pltpu.sync_copy(tmp_v, acc_out.at[pl.ds(c * NP_PAD + r0, 8)])

    return pl.kernel(
        body,
        out_type=jax.ShapeDtypeStruct((2 * NP_PAD, 176), jnp.float32),
        mesh=mesh,
        compiler_params=pltpu.CompilerParams(needs_layout_passes=False,
                                             use_tc_tiling_on_sc=False),
        scratch_types=(
            [pltpu.VMEM_SHARED((NP_PAD, 176), jnp.float32)]
            + [pltpu.VMEM((E_K,), jnp.int32),
               pltpu.VMEM((E_K,), jnp.int32),
               pltpu.VMEM((E_K,), jnp.int32),
               pltpu.VMEM((E_K, 176), jnp.float32),
               pltpu.VMEM((E_K, 16), jnp.float32),
               pltpu.VMEM((E_K,), jnp.int32)] * 3
            + [pltpu.VMEM((E_K * 16,), jnp.float32),
               pltpu.VMEM((8, 176), jnp.float32)]
            + [pltpu.SemaphoreType.DMA] * 9
        ),
    )


def _edge_sc(h2, aldp, srcp, dstp):
    N = aldp.shape[0]
    Epad = srcp.shape[0]
    h2f = h2.reshape(2 * N, 176)
    acc = _sc_edge_build(N, Epad)(h2f, aldp, srcp, dstp)
    acc = acc.reshape(2, NP_PAD, 176)
    return (acc[0, :N, :160], acc[1, :N, :160],
            acc[0, :N, 160:], acc[1, :N, 160:])


def kernel(x, edge_index, batch, c1_W, c1_as, c1_ad, c1_b, c2_W, c2_as, c2_ad,
           c2_b, c3_W, c3_as, c3_ad, c3_b, c4_W, c4_as, c4_ad, c4_b, c5_W,
           c5_as, c5_ad, c5_b, md1_W, md1_b, md2_W, md2_b, mh1_W, mh1_b,
           mh2_W, mh2_b, ln_w, ln_b):
    N = x.shape[0]
    loop = jnp.arange(N, dtype=edge_index.dtype)
    E_tot = edge_index.shape[1] + N
    Egrain = 16 * 3 * E_K
    Epad = ((E_tot + Egrain - 1) // Egrain) * Egrain
    padn = Epad - E_tot
    srcp = jnp.concatenate(
        [edge_index[0], loop, jnp.zeros((padn,), edge_index.dtype)])
    dstp = jnp.concatenate(
        [edge_index[1], loop,
         jnp.full((padn,), NP_PAD - 1, edge_index.dtype)])
    Mh, Me = _head_maps()

    layers = [
        (c1_W, c1_as, c1_ad, c1_b, md1_W, md1_b),
        (c2_W, c2_as, c2_ad, c2_b, md2_W, md2_b),
        (c3_W, c3_as, c3_ad, c3_b, None, None),
        (c4_W, c4_as, c4_ad, c4_b, None, None),
        (c5_W, c5_as, c5_ad, c5_b, None, None),
    ]

    xcur = x
    for (W, a_s, a_d, b, mdW, mdb) in layers:
        asv = a_s.reshape(1, HC)
        adv = a_d.reshape(1, HC)
        h2, aldp = _tca(xcur, W, asv, adv, Mh)
        accA, accB, denA, denB = _edge_sc(h2, aldp, srcp, dstp)
        xcur = _tcb(accA, accB, denA, denB, b, xcur, Me, mdW, mdb)

    bf = batch.astype(jnp.float32).reshape(N, 1)
    s, cnt = _pool(bf, xcur)
    return _headmlp(s, cnt, mh1_W, mh1_b, mh2_W, mh2_b, ln_w, ln_b)


# speedup vs baseline: 28.3522x; 1.0202x over previous
import functools

import jax
import jax.numpy as jnp
import numpy as np
from jax import lax
from jax.experimental import pallas as pl
from jax.experimental.pallas import tpu as pltpu

H_ = 5
C_ = 64
HC = 320
G_ = 64
NB_B = 1000
NP_PAD = 10016
E_K = 32


def _head_maps():
    Mh = np.zeros((320, 16), np.float32)
    Me = np.zeros((2, 16, 160), np.float32)
    for c in range(320):
        h = c // 64
        Mh[c, h] = 1.0
        Me[c // 160, h, c % 160] = 1.0
    return jnp.asarray(Mh), jnp.asarray(Me)


def _tca_body(x_ref, W_ref, asv_ref, adv_ref, M_ref, h2_ref, ald_ref):
    hb = jnp.dot(x_ref[...], W_ref[...], preferred_element_type=jnp.float32)
    als = jnp.dot(hb * asv_ref[...], M_ref[...],
                  preferred_element_type=jnp.float32)
    ald_ref[...] = jnp.dot(hb * adv_ref[...], M_ref[...],
                           preferred_element_type=jnp.float32)
    blk0 = jnp.concatenate([hb[:, :160], als], axis=-1)
    blk1 = jnp.concatenate([hb[:, 160:], als], axis=-1)
    h2_ref[...] = jnp.stack([blk0, blk1], axis=0)


def _tca(x, W, asv, adv, Mh):
    N, Din = x.shape
    B = NB_B
    return pl.pallas_call(
        _tca_body,
        grid=(N // B,),
        in_specs=[
            pl.BlockSpec((B, Din), lambda i: (i, 0)),
            pl.BlockSpec((Din, HC), lambda i: (0, 0)),
            pl.BlockSpec((1, HC), lambda i: (0, 0)),
            pl.BlockSpec((1, HC), lambda i: (0, 0)),
            pl.BlockSpec((HC, 16), lambda i: (0, 0)),
        ],
        out_specs=[
            pl.BlockSpec((2, B, 176), lambda i: (0, i, 0)),
            pl.BlockSpec((B, 16), lambda i: (i, 0)),
        ],
        out_shape=[
            jax.ShapeDtypeStruct((2, N, 176), jnp.float32),
            jax.ShapeDtypeStruct((N, 16), jnp.float32),
        ],
    )(x, W, asv, adv, Mh)


def _tcb_body_mlp(accA_r, accB_r, denA_r, denB_r, b_r, x_r, mdW_r, mdb_r,
                  Me_r, out_r):
    den16 = denA_r[...] + denB_r[...]
    dA = jnp.dot(den16, Me_r[0], preferred_element_type=jnp.float32)
    dB = jnp.dot(den16, Me_r[1], preferred_element_type=jnp.float32)
    msg = jnp.concatenate([accA_r[...] / dA, accB_r[...] / dB], axis=-1)
    x0 = jnp.maximum(msg + b_r[...], 0.0)
    out_r[...] = (jnp.dot(x_r[...], mdW_r[...],
                          preferred_element_type=jnp.float32)
                  + mdb_r[...] + x0)


def _tcb_body_plain(accA_r, accB_r, denA_r, denB_r, b_r, x_r, Me_r, out_r):
    den16 = denA_r[...] + denB_r[...]
    dA = jnp.dot(den16, Me_r[0], preferred_element_type=jnp.float32)
    dB = jnp.dot(den16, Me_r[1], preferred_element_type=jnp.float32)
    msg = jnp.concatenate([accA_r[...] / dA, accB_r[...] / dB], axis=-1)
    x0 = jnp.maximum(msg + b_r[...], 0.0)
    out_r[...] = x_r[...] + x0


def _tcb(accA, accB, denA, denB, b, x, Me, mdW=None, mdb=None):
    N = accA.shape[0]
    B = NB_B
    Din = x.shape[1]
    common_in = [
        pl.BlockSpec((B, 160), lambda i: (i, 0)),
        pl.BlockSpec((B, 160), lambda i: (i, 0)),
        pl.BlockSpec((B, 16), lambda i: (i, 0)),
        pl.BlockSpec((B, 16), lambda i: (i, 0)),
        pl.BlockSpec((1, HC), lambda i: (0, 0)),
        pl.BlockSpec((B, Din), lambda i: (i, 0)),
    ]
    if mdW is not None:
        in_specs = common_in + [
            pl.BlockSpec((Din, HC), lambda i: (0, 0)),
            pl.BlockSpec((1, HC), lambda i: (0, 0)),
            pl.BlockSpec((2, 16, 160), lambda i: (0, 0, 0)),
        ]
        args = (accA, accB, denA, denB, b.reshape(1, HC), x, mdW,
                mdb.reshape(1, HC), Me)
        body = _tcb_body_mlp
    else:
        in_specs = common_in + [pl.BlockSpec((2, 16, 160), lambda i: (0, 0, 0))]
        args = (accA, accB, denA, denB, b.reshape(1, HC), x, Me)
        body = _tcb_body_plain
    return pl.pallas_call(
        body,
        grid=(N // B,),
        in_specs=in_specs,
        out_specs=pl.BlockSpec((B, HC), lambda i: (i, 0)),
        out_shape=jax.ShapeDtypeStruct((N, HC), jnp.float32),
    )(*args)


def _pool_body(bf_r, x_r, s_r, cnt_r):
    i = pl.program_id(0)
    gids = lax.broadcasted_iota(jnp.int32, (1, G_), 1).astype(jnp.float32)
    mask = (bf_r[...] == gids).astype(jnp.float32)
    sp = lax.dot_general(mask, x_r[...], (((0,), (0,)), ((), ())),
                         preferred_element_type=jnp.float32)
    cp = lax.dot_general(mask, jnp.ones((mask.shape[0], 128), jnp.float32),
                         (((0,), (0,)), ((), ())),
                         preferred_element_type=jnp.float32)

    @pl.when(i == 0)
    def _():
        s_r[...] = sp
        cnt_r[...] = cp

    @pl.when(i != 0)
    def _():
        s_r[...] += sp
        cnt_r[...] += cp


def _pool(bf, x):
    N = x.shape[0]
    B = NB_B
    return pl.pallas_call(
        _pool_body,
        grid=(N // B,),
        in_specs=[
            pl.BlockSpec((B, 1), lambda i: (i, 0)),
            pl.BlockSpec((B, HC), lambda i: (i, 0)),
        ],
        out_specs=[
            pl.BlockSpec((G_, HC), lambda i: (0, 0)),
            pl.BlockSpec((G_, 128), lambda i: (0, 0)),
        ],
        out_shape=[
            jax.ShapeDtypeStruct((G_, HC), jnp.float32),
            jax.ShapeDtypeStruct((G_, 128), jnp.float32),
        ],
    )(bf, x)


def _head_body(s_r, cnt_r, w1_r, b1_r, w2_r, b2_r, lnw_r, lnb_r, out_r):
    cnt = jnp.maximum(cnt_r[:, 0:1], 1.0)
    xg = s_r[...] / cnt
    h1 = jnp.maximum(jnp.dot(xg, w1_r[...],
                             preferred_element_type=jnp.float32) + b1_r[...],
                     0.0)
    o = jnp.dot(h1, w2_r[...], preferred_element_type=jnp.float32) + b2_r[...]
    mu = jnp.mean(o, axis=-1, keepdims=True)
    var = jnp.mean((o - mu) ** 2, axis=-1, keepdims=True)
    out_r[...] = (o - mu) / jnp.sqrt(var + 1e-5) * lnw_r[...] + lnb_r[...]


def _headmlp(s, cnt, w1, b1, w2, b2, lnw, lnb):
    nhid = w1.shape[1]
    nout = w2.shape[1]
    return pl.pallas_call(
        _head_body,
        out_shape=jax.ShapeDtypeStruct((G_, nout), jnp.float32),
    )(s, cnt, w1, b1.reshape(1, nhid), w2, b2.reshape(1, nout),
      lnw.reshape(1, nout), lnb.reshape(1, nout))


@functools.lru_cache(maxsize=None)
def _sc_edge_build(N, Epad):
    from jax.experimental.pallas import tpu_sc as plsc

    NSUB = 16
    EPS = Epad // NSUB
    NCHUNK = EPS // E_K
    RPS = NP_PAD // NSUB
    WB_FULL = RPS // 8
    WB_TAIL = RPS - 8

    mesh = plsc.VectorSubcoreMesh(core_axis_name="c", subcore_axis_name="s")

    def body(h2_r, ald_r, src_r, dst_r, acc_out,
             acc_sh,
             src0, dst0, gidx0, rows0, ad0, sdst0,
             src1, dst1, gidx1, rows1, ad1, sdst1,
             src2, dst2, gidx2, rows2, ad2, sdst2,
             w_v, tmp_v,
             gsem0, gsem1, gsem2, sr0, sr1, sr2, is0_, is1_, is2_):
        c = lax.axis_index("c")
        s = lax.axis_index("s")
        lane = lax.broadcasted_iota(jnp.int32, (16,), 0)
        zero = jnp.zeros((16,), jnp.float32)
        cb = jnp.zeros((16,), jnp.int32) + c
        is0 = cb == 0
        den_mask = jnp.where(is0, lane < 3, (lane >= 3) & (lane < 5))
        headv = [jnp.where(is0,
                           jnp.full((16,), v // 4, jnp.int32),
                           jnp.full((16,), (10 + v) // 4, jnp.int32))
                 for v in range(10)]
        cN16 = cb * N
        bufs = ((src0, dst0, gidx0, rows0, ad0, sdst0, gsem0, sr0, is0_),
                (src1, dst1, gidx1, rows1, ad1, sdst1, gsem1, sr1, is1_),
                (src2, dst2, gidx2, rows2, ad2, sdst2, gsem2, sr2, is2_))
        ebase = s * EPS
        emax = Epad - E_K

        for r in range(8):
            for v in range(11):
                tmp_v[r, pl.ds(v * 16, 16)] = zero

        def zbody(k, carry):
            r0 = s * RPS + k * 8
            pltpu.sync_copy(tmp_v, acc_sh.at[pl.ds(r0, 8)])
            return carry

        lax.fori_loop(0, WB_FULL, zbody, 0)
        pltpu.sync_copy(tmp_v, acc_sh.at[pl.ds(s * RPS + WB_TAIL, 8)])
        plsc.subcore_barrier()

        def idx_base(cidx):
            return jnp.minimum(ebase + cidx * E_K, emax)

        def issue_idx(B, cidx):
            base = idx_base(cidx)
            pltpu.async_copy(src_r.at[pl.ds(base, E_K)], B[0], B[8])
            pltpu.async_copy(dst_r.at[pl.ds(base, E_K)], B[1], B[8])

        def wait_idx(B, cidx):
            base = idx_base(cidx)
            pltpu.make_async_copy(src_r.at[pl.ds(base, E_K)], B[0],
                                  B[8]).wait()
            pltpu.make_async_copy(dst_r.at[pl.ds(base, E_K)], B[1],
                                  B[8]).wait()

        def issue_gathers(B):
            for jj in range(E_K // 16):
                B[2][pl.ds(jj * 16, 16)] = B[0][pl.ds(jj * 16, 16)] + cN16
            pltpu.async_copy(h2_r.at[B[2]], B[3], B[6])
            pltpu.async_copy(ald_r.at[B[1]], B[4], B[6])

        def wait_gathers(B):
            pltpu.make_async_copy(h2_r.at[B[2]], B[3], B[6]).wait()
            pltpu.make_async_copy(ald_r.at[B[1]], B[4], B[6]).wait()

        def wait_scatter(B):
            pltpu.make_async_copy(B[3], acc_sh.at[B[5]], B[7]).wait()

        def compute_and_scatter(B):
            src_v, dst_v, gidx_v, rows_v, ad_v, sdst_v = B[:6]

            def wbody2(t, carry):
                for u in range(2):
                    j = 2 * t + u
                    e = rows_v[j, pl.ds(160, 16)] + ad_v[j]
                    w = jnp.exp(jnp.maximum(e, 0.2 * e))
                    w_v[pl.ds(j * 16, 16)] = w
                    rows_v[j, pl.ds(160, 16)] = jnp.where(den_mask, w, 0.0)
                return carry

            lax.fori_loop(0, E_K // 2, wbody2, 0)
            for jj in range(E_K // 16):
                sdst_v[pl.ds(jj * 16, 16)] = dst_v[pl.ds(jj * 16, 16)]

            def sbody(t, carry):
                for u in range(2):
                    j = 2 * t + u
                    jb = j * 16
                    for v in range(10):
                        splat = plsc.load_gather(w_v, [headv[v] + jb])
                        r = rows_v[j, pl.ds(v * 16, 16)]
                        rows_v[j, pl.ds(v * 16, 16)] = r * splat
                return carry

            lax.fori_loop(0, E_K // 2, sbody, 0)
            pltpu.async_copy(rows_v, acc_sh.at[sdst_v], B[7], add=True)

        def step_rest(cidx, B, Bn, Bf):
            wait_idx(Bn, cidx + 1)
            issue_gathers(Bn)
            issue_idx(Bf, cidx + 2)
            wait_gathers(B)
            compute_and_scatter(B)

        base0 = idx_base(jnp.int32(0))
        pltpu.sync_copy(src_r.at[pl.ds(base0, E_K)], bufs[0][0])
        pltpu.sync_copy(dst_r.at[pl.ds(base0, E_K)], bufs[0][1])
        issue_gathers(bufs[0])
        issue_idx(bufs[1], jnp.int32(1))

        def ebody(t, carry):
            c0 = 3 * t

            @pl.when(t > 0)
            def _():
                wait_scatter(bufs[1])

            step_rest(c0, bufs[0], bufs[1], bufs[2])

            @pl.when(t > 0)
            def _():
                wait_scatter(bufs[2])

            step_rest(c0 + 1, bufs[1], bufs[2], bufs[0])
            wait_scatter(bufs[0])
            step_rest(c0 + 2, bufs[2], bufs[0], bufs[1])
            return carry

        lax.fori_loop(0, NCHUNK // 3, ebody, 0)

        wait_gathers(bufs[0])
        wait_idx(bufs[1], jnp.int32(NCHUNK + 1))
        wait_scatter(bufs[1])
        wait_scatter(bufs[2])
        plsc.subcore_barrier()

        def wbody(k, carry):
            r0 = s * RPS + k * 8
            pltpu.sync_copy(acc_sh.at[pl.ds(r0, 8)], tmp_v)
            pltpu.sync_copy(tmp_v, acc_out.at[pl.ds(c * NP_PAD + r0, 8)])
            return carry

        lax.fori_loop(0, WB_FULL, wbody, 0)
        r0 = s * RPS + WB_TAIL
        pltpu.sync_copy(acc_sh.at[pl.ds(r0, 8)], tmp_v)
        pltpu.sync_copy(tmp_v, acc_out.at[pl.ds(c * NP_PAD + r0, 8)])

    return pl.kernel(
        body,
        out_type=jax.ShapeDtypeStruct((2 * NP_PAD, 176), jnp.float32),
        mesh=mesh,
        compiler_params=pltpu.CompilerParams(needs_layout_passes=False,
                                             use_tc_tiling_on_sc=False),
        scratch_types=(
            [pltpu.VMEM_SHARED((NP_PAD, 176), jnp.float32)]
            + [pltpu.VMEM((E_K,), jnp.int32),
               pltpu.VMEM((E_K,), jnp.int32),
               pltpu.VMEM((E_K,), jnp.int32),
               pltpu.VMEM((E_K, 176), jnp.float32),
               pltpu.VMEM((E_K, 16), jnp.float32),
               pltpu.VMEM((E_K,), jnp.int32)] * 3
            + [pltpu.VMEM((E_K * 16,), jnp.float32),
               pltpu.VMEM((8, 176), jnp.float32)]
            + [pltpu.SemaphoreType.DMA] * 9
        ),
    )


def _edge_sc(h2, aldp, srcp, dstp):
    N = aldp.shape[0]
    Epad = srcp.shape[0]
    h2f = h2.reshape(2 * N, 176)
    acc = _sc_edge_build(N, Epad)(h2f, aldp, srcp, dstp)
    acc = acc.reshape(2, NP_PAD, 176)
    return (acc[0, :N, :160], acc[1, :N, :160],
            acc[0, :N, 160:], acc[1, :N, 160:])


def kernel(x, edge_index, batch, c1_W, c1_as, c1_ad, c1_b, c2_W, c2_as, c2_ad,
           c2_b, c3_W, c3_as, c3_ad, c3_b, c4_W, c4_as, c4_ad, c4_b, c5_W,
           c5_as, c5_ad, c5_b, md1_W, md1_b, md2_W, md2_b, mh1_W, mh1_b,
           mh2_W, mh2_b, ln_w, ln_b):
    N = x.shape[0]
    loop = jnp.arange(N, dtype=edge_index.dtype)
    E_tot = edge_index.shape[1] + N
    Egrain = 16 * 3 * E_K
    Epad = ((E_tot + Egrain - 1) // Egrain) * Egrain
    padn = Epad - E_tot
    srcp = jnp.concatenate(
        [edge_index[0], loop, jnp.zeros((padn,), edge_index.dtype)])
    dstp = jnp.concatenate(
        [edge_index[1], loop,
         jnp.full((padn,), NP_PAD - 1, edge_index.dtype)])
    Mh, Me = _head_maps()

    layers = [
        (c1_W, c1_as, c1_ad, c1_b, md1_W, md1_b),
        (c2_W, c2_as, c2_ad, c2_b, md2_W, md2_b),
        (c3_W, c3_as, c3_ad, c3_b, None, None),
        (c4_W, c4_as, c4_ad, c4_b, None, None),
        (c5_W, c5_as, c5_ad, c5_b, None, None),
    ]

    xcur = x
    for (W, a_s, a_d, b, mdW, mdb) in layers:
        asv = a_s.reshape(1, HC)
        adv = a_d.reshape(1, HC)
        h2, aldp = _tca(xcur, W, asv, adv, Mh)
        accA, accB, denA, denB = _edge_sc(h2, aldp, srcp, dstp)
        xcur = _tcb(accA, accB, denA, denB, b, xcur, Me, mdW, mdb)

    bf = batch.astype(jnp.float32).reshape(N, 1)
    s, cnt = _pool(bf, xcur)
    return _headmlp(s, cnt, mh1_W, mh1_b, mh2_W, mh2_b, ln_w, ln_b)


# speedup vs baseline: 28.3977x; 1.0016x over previous
import functools

import jax
import jax.numpy as jnp
import numpy as np
from jax import lax
from jax.experimental import pallas as pl
from jax.experimental.pallas import tpu as pltpu

H_ = 5
C_ = 64
HC = 320
G_ = 64
NB_B = 1000
NP_PAD = 10016
E_K = 32


def _head_maps():
    Mh = np.zeros((320, 16), np.float32)
    Me = np.zeros((2, 16, 160), np.float32)
    for c in range(320):
        h = c // 64
        Mh[c, h] = 1.0
        Me[c // 160, h, c % 160] = 1.0
    return jnp.asarray(Mh), jnp.asarray(Me)


def _tca_body(x_ref, W_ref, asv_ref, adv_ref, M_ref, h2_ref, ald_ref):
    hb = jnp.dot(x_ref[...], W_ref[...], preferred_element_type=jnp.float32)
    als = jnp.dot(hb * asv_ref[...], M_ref[...],
                  preferred_element_type=jnp.float32)
    ald_ref[...] = jnp.dot(hb * adv_ref[...], M_ref[...],
                           preferred_element_type=jnp.float32)
    blk0 = jnp.concatenate([hb[:, :160], als], axis=-1)
    blk1 = jnp.concatenate([hb[:, 160:], als], axis=-1)
    h2_ref[...] = jnp.stack([blk0, blk1], axis=0)


def _tca(x, W, asv, adv, Mh):
    N, Din = x.shape
    B = NB_B
    return pl.pallas_call(
        _tca_body,
        grid=(N // B,),
        in_specs=[
            pl.BlockSpec((B, Din), lambda i: (i, 0)),
            pl.BlockSpec((Din, HC), lambda i: (0, 0)),
            pl.BlockSpec((1, HC), lambda i: (0, 0)),
            pl.BlockSpec((1, HC), lambda i: (0, 0)),
            pl.BlockSpec((HC, 16), lambda i: (0, 0)),
        ],
        out_specs=[
            pl.BlockSpec((2, B, 176), lambda i: (0, i, 0)),
            pl.BlockSpec((B, 16), lambda i: (i, 0)),
        ],
        out_shape=[
            jax.ShapeDtypeStruct((2, N, 176), jnp.float32),
            jax.ShapeDtypeStruct((N, 16), jnp.float32),
        ],
    )(x, W, asv, adv, Mh)


def _tcb_body_mlp(accA_r, accB_r, denA_r, denB_r, b_r, x_r, mdW_r, mdb_r,
                  Me_r, out_r):
    den16 = denA_r[...] + denB_r[...]
    dA = jnp.dot(den16, Me_r[0], preferred_element_type=jnp.float32)
    dB = jnp.dot(den16, Me_r[1], preferred_element_type=jnp.float32)
    msg = jnp.concatenate([accA_r[...] / dA, accB_r[...] / dB], axis=-1)
    x0 = jnp.maximum(msg + b_r[...], 0.0)
    out_r[...] = (jnp.dot(x_r[...], mdW_r[...],
                          preferred_element_type=jnp.float32)
                  + mdb_r[...] + x0)


def _tcb_body_plain(accA_r, accB_r, denA_r, denB_r, b_r, x_r, Me_r, out_r):
    den16 = denA_r[...] + denB_r[...]
    dA = jnp.dot(den16, Me_r[0], preferred_element_type=jnp.float32)
    dB = jnp.dot(den16, Me_r[1], preferred_element_type=jnp.float32)
    msg = jnp.concatenate([accA_r[...] / dA, accB_r[...] / dB], axis=-1)
    x0 = jnp.maximum(msg + b_r[...], 0.0)
    out_r[...] = x_r[...] + x0


def _tcb(accA, accB, denA, denB, b, x, Me, mdW=None, mdb=None):
    N = accA.shape[0]
    B = NB_B
    Din = x.shape[1]
    common_in = [
        pl.BlockSpec((B, 160), lambda i: (i, 0)),
        pl.BlockSpec((B, 160), lambda i: (i, 0)),
        pl.BlockSpec((B, 16), lambda i: (i, 0)),
        pl.BlockSpec((B, 16), lambda i: (i, 0)),
        pl.BlockSpec((1, HC), lambda i: (0, 0)),
        pl.BlockSpec((B, Din), lambda i: (i, 0)),
    ]
    if mdW is not None:
        in_specs = common_in + [
            pl.BlockSpec((Din, HC), lambda i: (0, 0)),
            pl.BlockSpec((1, HC), lambda i: (0, 0)),
            pl.BlockSpec((2, 16, 160), lambda i: (0, 0, 0)),
        ]
        args = (accA, accB, denA, denB, b.reshape(1, HC), x, mdW,
                mdb.reshape(1, HC), Me)
        body = _tcb_body_mlp
    else:
        in_specs = common_in + [pl.BlockSpec((2, 16, 160), lambda i: (0, 0, 0))]
        args = (accA, accB, denA, denB, b.reshape(1, HC), x, Me)
        body = _tcb_body_plain
    return pl.pallas_call(
        body,
        grid=(N // B,),
        in_specs=in_specs,
        out_specs=pl.BlockSpec((B, HC), lambda i: (i, 0)),
        out_shape=jax.ShapeDtypeStruct((N, HC), jnp.float32),
    )(*args)


def _pool_body(bf_r, x_r, s_r, cnt_r):
    i = pl.program_id(0)
    gids = lax.broadcasted_iota(jnp.int32, (1, G_), 1).astype(jnp.float32)
    mask = (bf_r[...] == gids).astype(jnp.float32)
    sp = lax.dot_general(mask, x_r[...], (((0,), (0,)), ((), ())),
                         preferred_element_type=jnp.float32)
    cp = lax.dot_general(mask, jnp.ones((mask.shape[0], 128), jnp.float32),
                         (((0,), (0,)), ((), ())),
                         preferred_element_type=jnp.float32)

    @pl.when(i == 0)
    def _():
        s_r[...] = sp
        cnt_r[...] = cp

    @pl.when(i != 0)
    def _():
        s_r[...] += sp
        cnt_r[...] += cp


def _pool(bf, x):
    N = x.shape[0]
    B = NB_B
    return pl.pallas_call(
        _pool_body,
        grid=(N // B,),
        in_specs=[
            pl.BlockSpec((B, 1), lambda i: (i, 0)),
            pl.BlockSpec((B, HC), lambda i: (i, 0)),
        ],
        out_specs=[
            pl.BlockSpec((G_, HC), lambda i: (0, 0)),
            pl.BlockSpec((G_, 128), lambda i: (0, 0)),
        ],
        out_shape=[
            jax.ShapeDtypeStruct((G_, HC), jnp.float32),
            jax.ShapeDtypeStruct((G_, 128), jnp.float32),
        ],
    )(bf, x)


def _head_body(s_r, cnt_r, w1_r, b1_r, w2_r, b2_r, lnw_r, lnb_r, out_r):
    cnt = jnp.maximum(cnt_r[:, 0:1], 1.0)
    xg = s_r[...] / cnt
    h1 = jnp.maximum(jnp.dot(xg, w1_r[...],
                             preferred_element_type=jnp.float32) + b1_r[...],
                     0.0)
    o = jnp.dot(h1, w2_r[...], preferred_element_type=jnp.float32) + b2_r[...]
    mu = jnp.mean(o, axis=-1, keepdims=True)
    var = jnp.mean((o - mu) ** 2, axis=-1, keepdims=True)
    out_r[...] = (o - mu) / jnp.sqrt(var + 1e-5) * lnw_r[...] + lnb_r[...]


def _headmlp(s, cnt, w1, b1, w2, b2, lnw, lnb):
    nhid = w1.shape[1]
    nout = w2.shape[1]
    return pl.pallas_call(
        _head_body,
        out_shape=jax.ShapeDtypeStruct((G_, nout), jnp.float32),
    )(s, cnt, w1, b1.reshape(1, nhid), w2, b2.reshape(1, nout),
      lnw.reshape(1, nout), lnb.reshape(1, nout))


@functools.lru_cache(maxsize=None)
def _sc_edge_build(N, Epad):
    from jax.experimental.pallas import tpu_sc as plsc

    NSUB = 16
    EPS = Epad // NSUB
    NCHUNK = EPS // E_K
    RPS = NP_PAD // NSUB
    WB_FULL = RPS // 8
    WB_TAIL = RPS - 8

    mesh = plsc.VectorSubcoreMesh(core_axis_name="c", subcore_axis_name="s")

    def body(h2_r, ald_r, sd_r, acc_out,
             acc_sh,
             sd0, dst0, gidx0, rows0, ad0,
             sd1, dst1, gidx1, rows1, ad1,
             sd2, dst2, gidx2, rows2, ad2,
             w_v, tmp_v,
             gsem0, gsem1, gsem2, sr0, sr1, sr2, is0_, is1_, is2_):
        c = lax.axis_index("c")
        s = lax.axis_index("s")
        lane = lax.broadcasted_iota(jnp.int32, (16,), 0)
        zero = jnp.zeros((16,), jnp.float32)
        cb = jnp.zeros((16,), jnp.int32) + c
        is0 = cb == 0
        den_mask = jnp.where(is0, lane < 3, (lane >= 3) & (lane < 5))
        headv = [jnp.where(is0,
                           jnp.full((16,), v // 4, jnp.int32),
                           jnp.full((16,), (10 + v) // 4, jnp.int32))
                 for v in range(10)]
        cN16 = cb * N
        bufs = ((sd0, dst0, gidx0, rows0, ad0, gsem0, sr0, is0_),
                (sd1, dst1, gidx1, rows1, ad1, gsem1, sr1, is1_),
                (sd2, dst2, gidx2, rows2, ad2, gsem2, sr2, is2_))
        ebase = s * EPS
        emax = Epad - E_K

        for r in range(8):
            for v in range(11):
                tmp_v[r, pl.ds(v * 16, 16)] = zero

        def zbody(k, carry):
            r0 = s * RPS + k * 8
            pltpu.sync_copy(tmp_v, acc_sh.at[pl.ds(r0, 8)])
            return carry

        lax.fori_loop(0, WB_FULL, zbody, 0)
        pltpu.sync_copy(tmp_v, acc_sh.at[pl.ds(s * RPS + WB_TAIL, 8)])
        plsc.subcore_barrier()

        def idx_base(cidx):
            return jnp.minimum(ebase + cidx * E_K, emax)

        def issue_idx(B, cidx):
            base = idx_base(cidx) * 2
            pltpu.async_copy(sd_r.at[pl.ds(base, 2 * E_K)], B[0], B[7])

        def wait_idx(B, cidx):
            base = idx_base(cidx) * 2
            pltpu.make_async_copy(sd_r.at[pl.ds(base, 2 * E_K)], B[0],
                                  B[7]).wait()

        def issue_gathers(B):
            for jj in range(E_K // 16):
                B[2][pl.ds(jj * 16, 16)] = B[0][pl.ds(jj * 16, 16)] + cN16
                B[1][pl.ds(jj * 16, 16)] = B[0][pl.ds(E_K + jj * 16, 16)]
            pltpu.async_copy(h2_r.at[B[2]], B[3], B[5])
            pltpu.async_copy(ald_r.at[B[1]], B[4], B[5])

        def wait_gathers(B):
            pltpu.make_async_copy(h2_r.at[B[2]], B[3], B[5]).wait()
            pltpu.make_async_copy(ald_r.at[B[1]], B[4], B[5]).wait()

        def wait_scatter(B):
            pltpu.make_async_copy(B[3], acc_sh.at[B[1]], B[6]).wait()

        def compute_and_scatter(B):
            sd_v, dst_v, gidx_v, rows_v, ad_v = B[:5]

            def wbody2(t, carry):
                for u in range(2):
                    j = 2 * t + u
                    e = rows_v[j, pl.ds(160, 16)] + ad_v[j]
                    w = jnp.exp(jnp.maximum(e, 0.2 * e))
                    w_v[pl.ds(j * 16, 16)] = w
                    rows_v[j, pl.ds(160, 16)] = jnp.where(den_mask, w, 0.0)
                return carry

            lax.fori_loop(0, E_K // 2, wbody2, 0)

            def sbody(t, carry):
                for u in range(2):
                    j = 2 * t + u
                    jb = j * 16
                    for v in range(10):
                        splat = plsc.load_gather(w_v, [headv[v] + jb])
                        r = rows_v[j, pl.ds(v * 16, 16)]
                        rows_v[j, pl.ds(v * 16, 16)] = r * splat
                return carry

            lax.fori_loop(0, E_K // 2, sbody, 0)
            pltpu.async_copy(rows_v, acc_sh.at[dst_v], B[6], add=True)

        def step_rest(cidx, B, Bn, Bf):
            wait_idx(Bn, cidx + 1)
            issue_gathers(Bn)
            issue_idx(Bf, cidx + 2)
            wait_gathers(B)
            compute_and_scatter(B)

        base0 = idx_base(jnp.int32(0)) * 2
        pltpu.sync_copy(sd_r.at[pl.ds(base0, 2 * E_K)], bufs[0][0])
        issue_gathers(bufs[0])
        issue_idx(bufs[1], jnp.int32(1))

        def ebody(t, carry):
            c0 = 3 * t

            @pl.when(t > 0)
            def _():
                wait_scatter(bufs[1])

            step_rest(c0, bufs[0], bufs[1], bufs[2])

            @pl.when(t > 0)
            def _():
                wait_scatter(bufs[2])

            step_rest(c0 + 1, bufs[1], bufs[2], bufs[0])
            wait_scatter(bufs[0])
            step_rest(c0 + 2, bufs[2], bufs[0], bufs[1])
            return carry

        lax.fori_loop(0, NCHUNK // 3, ebody, 0)

        wait_gathers(bufs[0])
        wait_idx(bufs[1], jnp.int32(NCHUNK + 1))
        wait_scatter(bufs[1])
        wait_scatter(bufs[2])
        plsc.subcore_barrier()

        def wbody(k, carry):
            r0 = s * RPS + k * 8
            pltpu.sync_copy(acc_sh.at[pl.ds(r0, 8)], tmp_v)
            pltpu.sync_copy(tmp_v, acc_out.at[pl.ds(c * NP_PAD + r0, 8)])
            return carry

        lax.fori_loop(0, WB_FULL, wbody, 0)
        r0 = s * RPS + WB_TAIL
        pltpu.sync_copy(acc_sh.at[pl.ds(r0, 8)], tmp_v)
        pltpu.sync_copy(tmp_v, acc_out.at[pl.ds(c * NP_PAD + r0, 8)])

    return pl.kernel(
        body,
        out_type=jax.ShapeDtypeStruct((2 * NP_PAD, 176), jnp.float32),
        mesh=mesh,
        compiler_params=pltpu.CompilerParams(needs_layout_passes=False,
                                             use_tc_tiling_on_sc=False),
        scratch_types=(
            [pltpu.VMEM_SHARED((NP_PAD, 176), jnp.float32)]
            + [pltpu.VMEM((2 * E_K,), jnp.int32),
               pltpu.VMEM((E_K,), jnp.int32),
               pltpu.VMEM((E_K,), jnp.int32),
               pltpu.VMEM((E_K, 176), jnp.float32),
               pltpu.VMEM((E_K, 16), jnp.float32)] * 3
            + [pltpu.VMEM((E_K * 16,), jnp.float32),
               pltpu.VMEM((8, 176), jnp.float32)]
            + [pltpu.SemaphoreType.DMA] * 9
        ),
    )


def _edge_sc(h2, aldp, sdp):
    N = aldp.shape[0]
    Epad = sdp.shape[0] // 2
    h2f = h2.reshape(2 * N, 176)
    acc = _sc_edge_build(N, Epad)(h2f, aldp, sdp)
    acc = acc.reshape(2, NP_PAD, 176)
    return (acc[0, :N, :160], acc[1, :N, :160],
            acc[0, :N, 160:], acc[1, :N, 160:])


def kernel(x, edge_index, batch, c1_W, c1_as, c1_ad, c1_b, c2_W, c2_as, c2_ad,
           c2_b, c3_W, c3_as, c3_ad, c3_b, c4_W, c4_as, c4_ad, c4_b, c5_W,
           c5_as, c5_ad, c5_b, md1_W, md1_b, md2_W, md2_b, mh1_W, mh1_b,
           mh2_W, mh2_b, ln_w, ln_b):
    N = x.shape[0]
    loop = jnp.arange(N, dtype=edge_index.dtype)
    E_tot = edge_index.shape[1] + N
    Egrain = 16 * 3 * E_K
    Epad = ((E_tot + Egrain - 1) // Egrain) * Egrain
    padn = Epad - E_tot
    srcp = jnp.concatenate(
        [edge_index[0], loop, jnp.zeros((padn,), edge_index.dtype)])
    dstp = jnp.concatenate(
        [edge_index[1], loop,
         jnp.full((padn,), NP_PAD - 1, edge_index.dtype)])
    sdp = jnp.concatenate([srcp.reshape(-1, E_K), dstp.reshape(-1, E_K)],
                          axis=1).reshape(-1)
    Mh, Me = _head_maps()

    layers = [
        (c1_W, c1_as, c1_ad, c1_b, md1_W, md1_b),
        (c2_W, c2_as, c2_ad, c2_b, md2_W, md2_b),
        (c3_W, c3_as, c3_ad, c3_b, None, None),
        (c4_W, c4_as, c4_ad, c4_b, None, None),
        (c5_W, c5_as, c5_ad, c5_b, None, None),
    ]

    xcur = x
    for (W, a_s, a_d, b, mdW, mdb) in layers:
        asv = a_s.reshape(1, HC)
        adv = a_d.reshape(1, HC)
        h2, aldp = _tca(xcur, W, asv, adv, Mh)
        accA, accB, denA, denB = _edge_sc(h2, aldp, sdp)
        xcur = _tcb(accA, accB, denA, denB, b, xcur, Me, mdW, mdb)

    bf = batch.astype(jnp.float32).reshape(N, 1)
    s, cnt = _pool(bf, xcur)
    return _headmlp(s, cnt, mh1_W, mh1_b, mh2_W, mh2_b, ln_w, ln_b)
